# Initial kernel scaffold; baseline (speedup 1.0000x reference)
import jax
import jax.numpy as jnp
from jax.experimental import pallas as pl


def kernel(adj_row, adj_col, adj_val, user_batch, rating_batch, item_batch, flag_t, users_features, gcn_users_embedding0, gcn_items_embedding0, f1_w1, f1_b1, f1_w2, f1_b2, f2_w1, f2_b1, f2_w2, f2_b2, d1_w1, d1_b1, d1_w2, d1_b2, d2_w1, d2_b1, d2_w2, d2_b2):
    raise NotImplementedError("write your pallas kernel here")


# speedup vs baseline: 5.8369x; 5.8369x over previous
import functools

import jax
import jax.numpy as jnp
from jax import lax
from jax.experimental import pallas as pl
from jax.experimental.pallas import tpu as pltpu
from jax.experimental.pallas import tpu_sc as plsc

UN = 50000
IN_ = 20000
F = 64
BN = 4096
E = 800000

NC, NS = 2, 16
NW = NC * NS
BPW = BN // NW
CH = 128
EPT = 50176
EPAD = EPT * NS
NCHUNK = EPT // CH
UNP = 50048
PERT = UNP // NS
HALF = F // 2


def _leaky(x):
    return jnp.where(x > 0, x, 0.01 * x)


def _item_mlp_body(x_ref, w1_ref, b1_ref, w2_ref, b2_ref, lo_ref, hi_ref):
    x = x_ref[...]
    h = _leaky(jnp.dot(x, w1_ref[...], preferred_element_type=jnp.float32)
               + b1_ref[...])
    o = (jnp.dot(h, w2_ref[...], preferred_element_type=jnp.float32)
         + b2_ref[...]) * 0.5
    lo_ref[...] = o[:, :HALF]
    hi_ref[...] = o[:, HALF:]


def _item_mlp(items, w1c, b1c, w2c, b2s):
    blk = 2000
    grid = IN_ // blk
    return pl.pallas_call(
        _item_mlp_body,
        grid=(grid,),
        in_specs=[
            pl.BlockSpec((blk, F), lambda i: (i, 0)),
            pl.BlockSpec((F, 4 * F), lambda i: (0, 0)),
            pl.BlockSpec((1, 4 * F), lambda i: (0, 0)),
            pl.BlockSpec((4 * F, F), lambda i: (0, 0)),
            pl.BlockSpec((1, F), lambda i: (0, 0)),
        ],
        out_specs=[
            pl.BlockSpec((blk, HALF), lambda i: (i, 0)),
            pl.BlockSpec((blk, HALF), lambda i: (i, 0)),
        ],
        out_shape=[
            jax.ShapeDtypeStruct((IN_, HALF), jnp.float32),
            jax.ShapeDtypeStruct((IN_, HALF), jnp.float32),
        ],
    )(items, w1c, b1c, w2c, b2s)


def _batch_body(ue_ref, fb_ref, ib_ref, rb_ref, w1_ref, b1_ref, w2_ref,
                b2_ref, wd1_ref, bd1_ref, wd2_ref, bd2_ref, out_ref):
    x = ue_ref[...]
    h = _leaky(jnp.dot(x, w1_ref[...], preferred_element_type=jnp.float32)
               + b1_ref[...])
    ub = (jnp.dot(h, w2_ref[...], preferred_element_type=jnp.float32)
          + b2_ref[...]) * 0.5
    hd = _leaky(jnp.dot(ub, wd1_ref[...], preferred_element_type=jnp.float32)
                + bd1_ref[...])
    logits = (jnp.dot(hd, wd2_ref[...], preferred_element_type=jnp.float32)
              + bd2_ref[...])
    col = lax.broadcasted_iota(jnp.int32, logits.shape, 1)
    neg = jnp.float32(-1e30)
    lse1 = jnp.log(jnp.sum(jnp.exp(jnp.where(col < 2, logits, neg)), -1))
    lse2 = jnp.log(jnp.sum(
        jnp.exp(jnp.where((col >= 2) & (col < 5), logits, neg)), -1))
    gender = fb_ref[...][:, 0:1]
    age = fb_ref[...][:, 1:2]
    ll1 = jnp.sum(jnp.where(col == gender, logits, 0.0), -1)
    ll2 = jnp.sum(jnp.where(col == 2 + age, logits, 0.0), -1)
    d_loss1 = jnp.mean(lse1 - ll1)
    d_loss2 = jnp.mean(lse2 - ll2)
    ib = ib_ref[...]
    pred = jnp.sum(ub * ib, -1)
    loss_part = jnp.mean((pred - rb_ref[...][:, 0]) ** 2)
    l2 = 0.001 * jnp.mean(jnp.sum(ub * ub + ib * ib, -1))
    lps = loss_part + l2
    ocol = lax.broadcasted_iota(jnp.int32, (1, 128), 1)
    out_ref[...] = (jnp.where(ocol == 0, d_loss1, 0.0)
                    + jnp.where(ocol == 1, d_loss2, 0.0)
                    + jnp.where(ocol == 2, lps, 0.0))


def _batch_losses(ue, fb, ib, rb2, w1c, b1c, w2c, b2s, wd1, bd1, wd2, bd2):
    return pl.pallas_call(
        _batch_body,
        in_specs=[pl.BlockSpec(a.shape, lambda: tuple(0 for _ in a.shape))
                  for a in (ue, fb, ib, rb2, w1c, b1c, w2c, b2s, wd1, bd1,
                            wd2, bd2)],
        out_specs=pl.BlockSpec((1, 128), lambda: (0, 0)),
        out_shape=jax.ShapeDtypeStruct((1, 128), jnp.float32),
    )(ue, fb, ib, rb2, w1c, b1c, w2c, b2s, wd1, bd1, wd2, bd2)


_LBLK = 3128


def _local_body(lo_ref, hi_ref, ft_ref, wd1_ref, bd1_ref, wd2_ref, bd2_ref,
                out_ref):
    i = pl.program_id(0)
    ng = pl.num_programs(0)
    x = jnp.concatenate([lo_ref[...], hi_ref[...]], axis=-1)
    hd = _leaky(jnp.dot(x, wd1_ref[...], preferred_element_type=jnp.float32)
                + bd1_ref[...])
    logits = (jnp.dot(hd, wd2_ref[...], preferred_element_type=jnp.float32)
              + bd2_ref[...])
    col = lax.broadcasted_iota(jnp.int32, logits.shape, 1)
    neg = jnp.float32(-1e30)
    lse1 = jnp.log(jnp.sum(jnp.exp(jnp.where(col < 2, logits, neg)), -1))
    lse2 = jnp.log(jnp.sum(
        jnp.exp(jnp.where((col >= 2) & (col < 5), logits, neg)), -1))
    gender = ft_ref[...][:, 0:1]
    age = ft_ref[...][:, 1:2]
    ll1 = jnp.sum(jnp.where(col == gender, logits, 0.0), -1)
    ll2 = jnp.sum(jnp.where(col == 2 + age, logits, 0.0), -1)
    ridx = i * _LBLK + lax.broadcasted_iota(jnp.int32, (logits.shape[0],), 0)
    valid = (ridx < UN).astype(jnp.float32)
    s1 = jnp.sum((lse1 - ll1) * valid)
    s2 = jnp.sum((lse2 - ll2) * valid)
    ocol = lax.broadcasted_iota(jnp.int32, (1, 128), 1)
    part = (jnp.where(ocol == 0, s1, 0.0) + jnp.where(ocol == 1, s2, 0.0))

    @pl.when(i == 0)
    def _():
        out_ref[...] = jnp.zeros_like(out_ref)

    out_ref[...] += part

    @pl.when(i == ng - 1)
    def _():
        out_ref[...] = out_ref[...] * (1.0 / UN)


def _local_losses(acc_lo, acc_hi, feats, wd1, bd1, wd2, bd2):
    blk = _LBLK
    grid = UNP // blk
    return pl.pallas_call(
        _local_body,
        grid=(grid,),
        in_specs=[
            pl.BlockSpec((blk, HALF), lambda i: (i, 0)),
            pl.BlockSpec((blk, HALF), lambda i: (i, 0)),
            pl.BlockSpec((blk, 16), lambda i: (i, 0)),
            pl.BlockSpec((F, 2 * F), lambda i: (0, 0)),
            pl.BlockSpec((1, 2 * F), lambda i: (0, 0)),
            pl.BlockSpec((2 * F, 128), lambda i: (0, 0)),
            pl.BlockSpec((1, 128), lambda i: (0, 0)),
        ],
        out_specs=pl.BlockSpec((1, 128), lambda i: (0, 0)),
        out_shape=jax.ShapeDtypeStruct((1, 128), jnp.float32),
    )(acc_lo, acc_hi, feats, wd1, bd1, wd2, bd2)


def _sc_body(adj_row, adj_col, adj_val, item_lo, item_hi, users_emb,
             user_batch, item_batch, feats,
             out_lo, out_hi, out_ue, out_iblo, out_ibhi, out_fb,
             col_v, row_v, val_v, rows_v, bidx_v, brow_v, bfeat_v, zero_v,
             acc_sh, sem):
    cid = lax.axis_index("c")
    sid = lax.axis_index("s")
    wid = sid * NC + cid

    abase = wid * BPW
    pltpu.sync_copy(user_batch.at[pl.ds(abase, BPW)], bidx_v)
    pltpu.async_copy(users_emb.at[bidx_v], brow_v, sem).wait()
    pltpu.sync_copy(brow_v, out_ue.at[pl.ds(abase, BPW)])
    pltpu.async_copy(feats.at[bidx_v], bfeat_v, sem).wait()
    pltpu.sync_copy(bfeat_v, out_fb.at[pl.ds(abase, BPW)])
    pltpu.sync_copy(item_batch.at[pl.ds(abase, BPW)], bidx_v)
    pltpu.async_copy(item_lo.at[bidx_v], rows_v, sem).wait()
    pltpu.sync_copy(rows_v, out_iblo.at[pl.ds(abase, BPW)])
    pltpu.async_copy(item_hi.at[bidx_v], rows_v, sem).wait()
    pltpu.sync_copy(rows_v, out_ibhi.at[pl.ds(abase, BPW)])

    zeros16 = jnp.zeros((16,), jnp.float32)
    for i in range(128):
        zero_v[i, pl.ds(0, 16)] = zeros16
        zero_v[i, pl.ds(16, 16)] = zeros16
    zbase = sid * PERT
    nfull = PERT // 128
    rem = PERT - nfull * 128
    for z in range(nfull):
        pltpu.sync_copy(zero_v, acc_sh.at[pl.ds(zbase + z * 128, 128)])
    pltpu.sync_copy(zero_v.at[pl.ds(0, rem)],
                    acc_sh.at[pl.ds(zbase + nfull * 128, rem)])
    plsc.subcore_barrier()

    tbase = sid * EPT

    def chunk(k, carry):
        base = tbase + k * CH
        pltpu.sync_copy(adj_col.at[pl.ds(base, CH)], col_v)
        pltpu.sync_copy(adj_val.at[pl.ds(base, CH)], val_v)
        pltpu.sync_copy(adj_row.at[pl.ds(base, CH)], row_v)

        @pl.when(cid == 0)
        def _():
            pltpu.async_copy(item_lo.at[col_v], rows_v, sem).wait()

        @pl.when(cid == 1)
        def _():
            pltpu.async_copy(item_hi.at[col_v], rows_v, sem).wait()

        for g in range(CH // 16):
            val16 = val_v[pl.ds(g * 16, 16)]
            for j in range(16):
                e = g * 16 + j
                vv = jnp.broadcast_to(lax.slice(val16, (j,), (j + 1,)), (16,))
                rows_v[e, pl.ds(0, 16)] = rows_v[e, pl.ds(0, 16)] * vv
                rows_v[e, pl.ds(16, 16)] = rows_v[e, pl.ds(16, 16)] * vv
        pltpu.sync_copy(rows_v, acc_sh.at[row_v], add=True)
        return carry

    lax.fori_loop(0, NCHUNK, chunk, 0)
    plsc.subcore_barrier()

    dbase = sid * PERT

    @pl.when(cid == 0)
    def _():
        pltpu.sync_copy(acc_sh.at[pl.ds(dbase, PERT)],
                        out_lo.at[pl.ds(dbase, PERT)])

    @pl.when(cid == 1)
    def _():
        pltpu.sync_copy(acc_sh.at[pl.ds(dbase, PERT)],
                        out_hi.at[pl.ds(dbase, PERT)])


def _sc_run(adj_row_p, adj_col_p, adj_val_p, item_lo, item_hi, users_emb,
            user_batch, item_batch, feats_p):
    mesh = plsc.VectorSubcoreMesh(core_axis_name="c", subcore_axis_name="s",
                                  num_cores=NC, num_subcores=NS)
    f = functools.partial(
        pl.kernel,
        out_type=(
            jax.ShapeDtypeStruct((UNP, HALF), jnp.float32),
            jax.ShapeDtypeStruct((UNP, HALF), jnp.float32),
            jax.ShapeDtypeStruct((BN, F), jnp.float32),
            jax.ShapeDtypeStruct((BN, HALF), jnp.float32),
            jax.ShapeDtypeStruct((BN, HALF), jnp.float32),
            jax.ShapeDtypeStruct((BN, 16), jnp.int32),
        ),
        mesh=mesh,
        scratch_types=[
            pltpu.VMEM((CH,), jnp.int32),
            pltpu.VMEM((CH,), jnp.int32),
            pltpu.VMEM((CH,), jnp.float32),
            pltpu.VMEM((CH, HALF), jnp.float32),
            pltpu.VMEM((BPW,), jnp.int32),
            pltpu.VMEM((BPW, F), jnp.float32),
            pltpu.VMEM((BPW, 16), jnp.int32),
            pltpu.VMEM((128, HALF), jnp.float32),
            pltpu.VMEM_SHARED((UNP, HALF), jnp.float32),
            pltpu.SemaphoreType.DMA,
        ],
        compiler_params=pltpu.CompilerParams(use_tc_tiling_on_sc=False),
    )(_sc_body)
    return f(adj_row_p, adj_col_p, adj_val_p, item_lo, item_hi, users_emb,
             user_batch, item_batch, feats_p)


def kernel(adj_row, adj_col, adj_val, user_batch, rating_batch, item_batch,
           flag_t, users_features, gcn_users_embedding0, gcn_items_embedding0,
           f1_w1, f1_b1, f1_w2, f1_b2, f2_w1, f2_b1, f2_w2, f2_b2,
           d1_w1, d1_b1, d1_w2, d1_b2, d2_w1, d2_b1, d2_w2, d2_b2):
    w1c = jnp.concatenate([f1_w1, f2_w1], axis=1)
    b1c = jnp.concatenate([f1_b1, f2_b1])[None, :]
    w2c = jnp.concatenate([f1_w2, f2_w2], axis=0)
    b2s = (f1_b2 + f2_b2)[None, :]
    wd1 = jnp.concatenate([d1_w1, d2_w1], axis=1)
    bd1 = jnp.concatenate([d1_b1, d2_b1])[None, :]
    wd2 = jnp.zeros((2 * F, 128), jnp.float32)
    wd2 = wd2.at[:F, 0:2].set(d1_w2).at[F:, 2:5].set(d2_w2)
    bd2 = jnp.zeros((128,), jnp.float32)
    bd2 = bd2.at[0:2].set(d1_b2).at[2:5].set(d2_b2)[None, :]

    pe = EPAD - E
    adj_row_p = jnp.concatenate([adj_row, jnp.zeros((pe,), jnp.int32)])
    adj_col_p = jnp.concatenate([adj_col, jnp.zeros((pe,), jnp.int32)])
    adj_val_p = jnp.concatenate([adj_val, jnp.zeros((pe,), jnp.float32)])
    feats_p = jnp.pad(users_features, ((0, UNP - UN), (0, 14)))

    item_lo, item_hi = _item_mlp(gcn_items_embedding0, w1c, b1c, w2c, b2s)

    acc_lo, acc_hi, ue, ib_lo, ib_hi, fb = _sc_run(
        adj_row_p, adj_col_p, adj_val_p, item_lo, item_hi,
        gcn_users_embedding0, user_batch, item_batch, feats_p)

    ib = jnp.concatenate([ib_lo, ib_hi], axis=1)
    rb2 = rating_batch[:, None]

    bout = _batch_losses(ue, fb, ib, rb2, w1c, b1c, w2c, b2s,
                         wd1, bd1, wd2, bd2)
    lout = _local_losses(acc_lo, acc_hi, feats_p, wd1, bd1, wd2, bd2)

    d_loss1 = bout[0, 0]
    d_loss2 = bout[0, 1]
    lps = bout[0, 2]
    d_loss1_local = lout[0, 0]
    d_loss2_local = lout[0, 1]

    d_loss = (d_loss1 * 2.0 + d_loss2) / 2.0
    d_loss_local = d_loss1_local * 2.0 + d_loss2_local
    d_loss_all = 10.0 * (d_loss + 0.5 * d_loss_local)
    g_loss_all = 0.1 * lps - d_loss_all
    g_d_loss_all = -d_loss_all
    return (d_loss_all, g_loss_all, g_d_loss_all)


# speedup vs baseline: 12.8282x; 2.1978x over previous
import functools

import jax
import jax.numpy as jnp
from jax import lax
from jax.experimental import pallas as pl
from jax.experimental.pallas import tpu as pltpu
from jax.experimental.pallas import tpu_sc as plsc

UN = 50000
IN_ = 20000
F = 64
BN = 4096
E = 800000

NC, NS = 2, 16
NW = NC * NS
BPW = BN // NW
CH = 128
EPT = 50176
EPAD = EPT * NS
NCHUNK = EPT // CH
UNP = 50048
PERT = UNP // NS
HALF = F // 2


def _leaky(x):
    return jnp.where(x > 0, x, 0.01 * x)


def _item_mlp_body(x_ref, w1_ref, b1_ref, w2_ref, b2_ref, lo_ref, hi_ref):
    x = x_ref[...]
    h = _leaky(jnp.dot(x, w1_ref[...], preferred_element_type=jnp.float32)
               + b1_ref[...])
    o = (jnp.dot(h, w2_ref[...], preferred_element_type=jnp.float32)
         + b2_ref[...]) * 0.5
    lo_ref[...] = o[:, :HALF]
    hi_ref[...] = o[:, HALF:]


def _item_mlp(items, w1c, b1c, w2c, b2s):
    blk = 2000
    grid = IN_ // blk
    return pl.pallas_call(
        _item_mlp_body,
        grid=(grid,),
        in_specs=[
            pl.BlockSpec((blk, F), lambda i: (i, 0)),
            pl.BlockSpec((F, 4 * F), lambda i: (0, 0)),
            pl.BlockSpec((1, 4 * F), lambda i: (0, 0)),
            pl.BlockSpec((4 * F, F), lambda i: (0, 0)),
            pl.BlockSpec((1, F), lambda i: (0, 0)),
        ],
        out_specs=[
            pl.BlockSpec((blk, HALF), lambda i: (i, 0)),
            pl.BlockSpec((blk, HALF), lambda i: (i, 0)),
        ],
        out_shape=[
            jax.ShapeDtypeStruct((IN_, HALF), jnp.float32),
            jax.ShapeDtypeStruct((IN_, HALF), jnp.float32),
        ],
    )(items, w1c, b1c, w2c, b2s)


def _batch_body(ue_ref, fb_ref, ib_ref, rb_ref, w1_ref, b1_ref, w2_ref,
                b2_ref, wd1_ref, bd1_ref, wd2_ref, bd2_ref, out_ref):
    x = ue_ref[...]
    h = _leaky(jnp.dot(x, w1_ref[...], preferred_element_type=jnp.float32)
               + b1_ref[...])
    ub = (jnp.dot(h, w2_ref[...], preferred_element_type=jnp.float32)
          + b2_ref[...]) * 0.5
    hd = _leaky(jnp.dot(ub, wd1_ref[...], preferred_element_type=jnp.float32)
                + bd1_ref[...])
    logits = (jnp.dot(hd, wd2_ref[...], preferred_element_type=jnp.float32)
              + bd2_ref[...])
    col = lax.broadcasted_iota(jnp.int32, logits.shape, 1)
    neg = jnp.float32(-1e30)
    lse1 = jnp.log(jnp.sum(jnp.exp(jnp.where(col < 2, logits, neg)), -1))
    lse2 = jnp.log(jnp.sum(
        jnp.exp(jnp.where((col >= 2) & (col < 5), logits, neg)), -1))
    gender = fb_ref[...][:, 0:1]
    age = fb_ref[...][:, 1:2]
    ll1 = jnp.sum(jnp.where(col == gender, logits, 0.0), -1)
    ll2 = jnp.sum(jnp.where(col == 2 + age, logits, 0.0), -1)
    d_loss1 = jnp.mean(lse1 - ll1)
    d_loss2 = jnp.mean(lse2 - ll2)
    ib = ib_ref[...]
    pred = jnp.sum(ub * ib, -1)
    loss_part = jnp.mean((pred - rb_ref[...][:, 0]) ** 2)
    l2 = 0.001 * jnp.mean(jnp.sum(ub * ub + ib * ib, -1))
    lps = loss_part + l2
    ocol = lax.broadcasted_iota(jnp.int32, (1, 128), 1)
    out_ref[...] = (jnp.where(ocol == 0, d_loss1, 0.0)
                    + jnp.where(ocol == 1, d_loss2, 0.0)
                    + jnp.where(ocol == 2, lps, 0.0))


def _batch_losses(ue, fb, ib, rb2, w1c, b1c, w2c, b2s, wd1, bd1, wd2, bd2):
    return pl.pallas_call(
        _batch_body,
        in_specs=[pl.BlockSpec(a.shape, lambda: tuple(0 for _ in a.shape))
                  for a in (ue, fb, ib, rb2, w1c, b1c, w2c, b2s, wd1, bd1,
                            wd2, bd2)],
        out_specs=pl.BlockSpec((1, 128), lambda: (0, 0)),
        out_shape=jax.ShapeDtypeStruct((1, 128), jnp.float32),
    )(ue, fb, ib, rb2, w1c, b1c, w2c, b2s, wd1, bd1, wd2, bd2)


_LBLK = 3128


def _local_body(lo_ref, hi_ref, ft_ref, wd1_ref, bd1_ref, wd2_ref, bd2_ref,
                out_ref):
    i = pl.program_id(0)
    ng = pl.num_programs(0)
    x = jnp.concatenate([lo_ref[...], hi_ref[...]], axis=-1)
    hd = _leaky(jnp.dot(x, wd1_ref[...], preferred_element_type=jnp.float32)
                + bd1_ref[...])
    logits = (jnp.dot(hd, wd2_ref[...], preferred_element_type=jnp.float32)
              + bd2_ref[...])
    col = lax.broadcasted_iota(jnp.int32, logits.shape, 1)
    neg = jnp.float32(-1e30)
    lse1 = jnp.log(jnp.sum(jnp.exp(jnp.where(col < 2, logits, neg)), -1))
    lse2 = jnp.log(jnp.sum(
        jnp.exp(jnp.where((col >= 2) & (col < 5), logits, neg)), -1))
    gender = ft_ref[...][:, 0:1]
    age = ft_ref[...][:, 1:2]
    ll1 = jnp.sum(jnp.where(col == gender, logits, 0.0), -1)
    ll2 = jnp.sum(jnp.where(col == 2 + age, logits, 0.0), -1)
    ridx = i * _LBLK + lax.broadcasted_iota(jnp.int32, (logits.shape[0],), 0)
    valid = (ridx < UN).astype(jnp.float32)
    s1 = jnp.sum((lse1 - ll1) * valid)
    s2 = jnp.sum((lse2 - ll2) * valid)
    ocol = lax.broadcasted_iota(jnp.int32, (1, 128), 1)
    part = (jnp.where(ocol == 0, s1, 0.0) + jnp.where(ocol == 1, s2, 0.0))

    @pl.when(i == 0)
    def _():
        out_ref[...] = jnp.zeros_like(out_ref)

    out_ref[...] += part

    @pl.when(i == ng - 1)
    def _():
        out_ref[...] = out_ref[...] * (1.0 / UN)


def _local_losses(acc_lo, acc_hi, feats, wd1, bd1, wd2, bd2):
    blk = _LBLK
    grid = UNP // blk
    return pl.pallas_call(
        _local_body,
        grid=(grid,),
        in_specs=[
            pl.BlockSpec((blk, HALF), lambda i: (i, 0)),
            pl.BlockSpec((blk, HALF), lambda i: (i, 0)),
            pl.BlockSpec((blk, 16), lambda i: (i, 0)),
            pl.BlockSpec((F, 2 * F), lambda i: (0, 0)),
            pl.BlockSpec((1, 2 * F), lambda i: (0, 0)),
            pl.BlockSpec((2 * F, 128), lambda i: (0, 0)),
            pl.BlockSpec((1, 128), lambda i: (0, 0)),
        ],
        out_specs=pl.BlockSpec((1, 128), lambda i: (0, 0)),
        out_shape=jax.ShapeDtypeStruct((1, 128), jnp.float32),
    )(acc_lo, acc_hi, feats, wd1, bd1, wd2, bd2)


QUAD = 4


def _sc_body(adj_row2, adj_col2, adj_val2, item_cat, users_emb,
             user_batch, item_batch, feats,
             out_lo, out_hi, out_ue, out_iblo, out_ibhi, out_fb,
             rows0, rows1, rows2, rows3,
             col0, col1, col2, col3, row0, row1, row2, row3,
             val0, val1, val2, val3,
             bidx_v, bidx2_v, brow_v, bfeat_v,
             acc_sh, sg, ss, se):
    cid = lax.axis_index("c")
    sid = lax.axis_index("s")
    wid = sid * NC + cid
    rows_bufs = (rows0, rows1, rows2, rows3)
    col_bufs = (col0, col1, col2, col3)
    row_bufs = (row0, row1, row2, row3)
    val_bufs = (val0, val1, val2, val3)

    abase = wid * BPW
    pltpu.sync_copy(user_batch.at[pl.ds(abase, BPW)], bidx_v)
    for p in range(BPW // 32):
        pltpu.async_copy(users_emb.at[bidx_v.at[pl.ds(p * 32, 32)]],
                         brow_v, sg.at[0]).wait()
        pltpu.sync_copy(brow_v, out_ue.at[pl.ds(abase + p * 32, 32)])
    for p in range(BPW // 64):
        pltpu.async_copy(feats.at[bidx_v.at[pl.ds(p * 64, 64)]],
                         bfeat_v, sg.at[0]).wait()
        pltpu.sync_copy(bfeat_v, out_fb.at[pl.ds(abase + p * 64, 64)])
    pltpu.sync_copy(item_batch.at[pl.ds(abase, BPW)], bidx_v)
    off16 = jnp.full((16,), IN_, jnp.int32)
    for g in range(BPW // 16):
        bidx2_v[pl.ds(g * 16, 16)] = bidx_v[pl.ds(g * 16, 16)] + off16
    pltpu.async_copy(item_cat.at[bidx_v], rows0, sg.at[0]).wait()
    pltpu.sync_copy(rows0, out_iblo.at[pl.ds(abase, BPW)])
    pltpu.async_copy(item_cat.at[bidx2_v], rows0, sg.at[0]).wait()
    pltpu.sync_copy(rows0, out_ibhi.at[pl.ds(abase, BPW)])

    zeros16 = jnp.zeros((16,), jnp.float32)
    for i in range(CH):
        rows0[i, pl.ds(0, 16)] = zeros16
        rows0[i, pl.ds(16, 16)] = zeros16
    for g in range(BPW // 16):
        bidx_v[pl.ds(g * 16, 16)] = jnp.zeros((16,), jnp.int32)
    zbase = sid * PERT
    nfull = PERT // CH
    rem = PERT - nfull * CH
    for z in range(nfull):
        pltpu.sync_copy(rows0, acc_sh.at[pl.ds(zbase + z * CH, CH)])
    pltpu.sync_copy(rows0.at[pl.ds(0, rem)],
                    acc_sh.at[pl.ds(zbase + nfull * CH, rem)])
    plsc.subcore_barrier()

    coff = jnp.broadcast_to(cid * IN_, (16,)).astype(jnp.int32)
    tchunk = sid * NCHUNK

    for b in range(1, QUAD):
        pltpu.sync_copy(acc_sh.at[pl.ds(zbase, CH)], rows_bufs[b])
    for b in range(QUAD):
        pltpu.async_copy(rows_bufs[b], acc_sh.at[bidx_v], ss.at[b], add=True)

    def quad(k, c):
        for b in range(QUAD):
            j = tchunk + k * QUAD + b
            pltpu.make_async_copy(rows_bufs[b], acc_sh.at[bidx_v],
                                  ss.at[b]).wait()
            pltpu.async_copy(adj_col2.at[j], col_bufs[b], se.at[b])
            pltpu.async_copy(adj_row2.at[j], row_bufs[b], se.at[b])
            pltpu.async_copy(adj_val2.at[j], val_bufs[b], se.at[b])
        for b in range(QUAD):
            j = tchunk + k * QUAD + b
            pltpu.make_async_copy(adj_col2.at[j], col_bufs[b], se.at[b]).wait()
            pltpu.make_async_copy(adj_row2.at[j], row_bufs[b], se.at[b]).wait()
            pltpu.make_async_copy(adj_val2.at[j], val_bufs[b], se.at[b]).wait()
            for g in range(CH // 16):
                col_bufs[b][pl.ds(g * 16, 16)] = (
                    col_bufs[b][pl.ds(g * 16, 16)] + coff)
            pltpu.async_copy(item_cat.at[col_bufs[b]], rows_bufs[b], sg.at[b])
        for b in range(QUAD):
            pltpu.make_async_copy(item_cat.at[col_bufs[b]], rows_bufs[b],
                                  sg.at[b]).wait()
            rb = rows_bufs[b]
            vb = val_bufs[b]
            for g in range(CH // 16):
                val16 = vb[pl.ds(g * 16, 16)]
                for t in range(16):
                    e = g * 16 + t
                    vv = jnp.broadcast_to(
                        lax.slice(val16, (t,), (t + 1,)), (16,))
                    rb[e, pl.ds(0, 16)] = rb[e, pl.ds(0, 16)] * vv
                    rb[e, pl.ds(16, 16)] = rb[e, pl.ds(16, 16)] * vv
            pltpu.async_copy(rb, acc_sh.at[row_bufs[b]], ss.at[b], add=True)
        return c

    lax.fori_loop(0, NCHUNK // QUAD, quad, 0)
    for b in range(QUAD):
        pltpu.make_async_copy(rows_bufs[b], acc_sh.at[bidx_v], ss.at[b]).wait()
    plsc.subcore_barrier()

    dbase = sid * PERT

    @pl.when(cid == 0)
    def _():
        pltpu.sync_copy(acc_sh.at[pl.ds(dbase, PERT)],
                        out_lo.at[pl.ds(dbase, PERT)])

    @pl.when(cid == 1)
    def _():
        pltpu.sync_copy(acc_sh.at[pl.ds(dbase, PERT)],
                        out_hi.at[pl.ds(dbase, PERT)])


def _sc_run(adj_row2, adj_col2, adj_val2, item_cat, users_emb,
            user_batch, item_batch, feats_p):
    mesh = plsc.VectorSubcoreMesh(core_axis_name="c", subcore_axis_name="s",
                                  num_cores=NC, num_subcores=NS)
    f = functools.partial(
        pl.kernel,
        out_type=(
            jax.ShapeDtypeStruct((UNP, HALF), jnp.float32),
            jax.ShapeDtypeStruct((UNP, HALF), jnp.float32),
            jax.ShapeDtypeStruct((BN, F), jnp.float32),
            jax.ShapeDtypeStruct((BN, HALF), jnp.float32),
            jax.ShapeDtypeStruct((BN, HALF), jnp.float32),
            jax.ShapeDtypeStruct((BN, 16), jnp.int32),
        ),
        mesh=mesh,
        scratch_types=[
            pltpu.VMEM((CH, HALF), jnp.float32),
            pltpu.VMEM((CH, HALF), jnp.float32),
            pltpu.VMEM((CH, HALF), jnp.float32),
            pltpu.VMEM((CH, HALF), jnp.float32),
            pltpu.VMEM((CH,), jnp.int32),
            pltpu.VMEM((CH,), jnp.int32),
            pltpu.VMEM((CH,), jnp.int32),
            pltpu.VMEM((CH,), jnp.int32),
            pltpu.VMEM((CH,), jnp.int32),
            pltpu.VMEM((CH,), jnp.int32),
            pltpu.VMEM((CH,), jnp.int32),
            pltpu.VMEM((CH,), jnp.int32),
            pltpu.VMEM((CH,), jnp.float32),
            pltpu.VMEM((CH,), jnp.float32),
            pltpu.VMEM((CH,), jnp.float32),
            pltpu.VMEM((CH,), jnp.float32),
            pltpu.VMEM((BPW,), jnp.int32),
            pltpu.VMEM((BPW,), jnp.int32),
            pltpu.VMEM((32, F), jnp.float32),
            pltpu.VMEM((64, 16), jnp.int32),
            pltpu.VMEM_SHARED((UNP, HALF), jnp.float32),
            pltpu.SemaphoreType.DMA((QUAD,)),
            pltpu.SemaphoreType.DMA((QUAD,)),
            pltpu.SemaphoreType.DMA((QUAD,)),
        ],
        compiler_params=pltpu.CompilerParams(use_tc_tiling_on_sc=False),
    )(_sc_body)
    return f(adj_row2, adj_col2, adj_val2, item_cat, users_emb,
             user_batch, item_batch, feats_p)


def kernel(adj_row, adj_col, adj_val, user_batch, rating_batch, item_batch,
           flag_t, users_features, gcn_users_embedding0, gcn_items_embedding0,
           f1_w1, f1_b1, f1_w2, f1_b2, f2_w1, f2_b1, f2_w2, f2_b2,
           d1_w1, d1_b1, d1_w2, d1_b2, d2_w1, d2_b1, d2_w2, d2_b2):
    w1c = jnp.concatenate([f1_w1, f2_w1], axis=1)
    b1c = jnp.concatenate([f1_b1, f2_b1])[None, :]
    w2c = jnp.concatenate([f1_w2, f2_w2], axis=0)
    b2s = (f1_b2 + f2_b2)[None, :]
    wd1 = jnp.concatenate([d1_w1, d2_w1], axis=1)
    bd1 = jnp.concatenate([d1_b1, d2_b1])[None, :]
    wd2 = jnp.zeros((2 * F, 128), jnp.float32)
    wd2 = wd2.at[:F, 0:2].set(d1_w2).at[F:, 2:5].set(d2_w2)
    bd2 = jnp.zeros((128,), jnp.float32)
    bd2 = bd2.at[0:2].set(d1_b2).at[2:5].set(d2_b2)[None, :]

    pe = EPAD - E
    adj_row2 = jnp.concatenate(
        [adj_row, jnp.zeros((pe,), jnp.int32)]).reshape(EPAD // CH, CH)
    adj_col2 = jnp.concatenate(
        [adj_col, jnp.zeros((pe,), jnp.int32)]).reshape(EPAD // CH, CH)
    adj_val2 = jnp.concatenate(
        [adj_val, jnp.zeros((pe,), jnp.float32)]).reshape(EPAD // CH, CH)
    feats_p = jnp.pad(users_features, ((0, UNP - UN), (0, 14)))

    item_lo, item_hi = _item_mlp(gcn_items_embedding0, w1c, b1c, w2c, b2s)
    item_cat = jnp.concatenate([item_lo, item_hi], axis=0)

    acc_lo, acc_hi, ue, ib_lo, ib_hi, fb = _sc_run(
        adj_row2, adj_col2, adj_val2, item_cat,
        gcn_users_embedding0, user_batch, item_batch, feats_p)

    ib = jnp.concatenate([ib_lo, ib_hi], axis=1)
    rb2 = rating_batch[:, None]

    bout = _batch_losses(ue, fb, ib, rb2, w1c, b1c, w2c, b2s,
                         wd1, bd1, wd2, bd2)
    lout = _local_losses(acc_lo, acc_hi, feats_p, wd1, bd1, wd2, bd2)

    d_loss1 = bout[0, 0]
    d_loss2 = bout[0, 1]
    lps = bout[0, 2]
    d_loss1_local = lout[0, 0]
    d_loss2_local = lout[0, 1]

    d_loss = (d_loss1 * 2.0 + d_loss2) / 2.0
    d_loss_local = d_loss1_local * 2.0 + d_loss2_local
    d_loss_all = 10.0 * (d_loss + 0.5 * d_loss_local)
    g_loss_all = 0.1 * lps - d_loss_all
    g_d_loss_all = -d_loss_all
    return (d_loss_all, g_loss_all, g_d_loss_all)


# speedup vs baseline: 13.6154x; 1.0614x over previous
import functools

import jax
import jax.numpy as jnp
from jax import lax
from jax.experimental import pallas as pl
from jax.experimental.pallas import tpu as pltpu
from jax.experimental.pallas import tpu_sc as plsc

UN = 50000
IN_ = 20000
F = 64
BN = 4096
E = 800000

NC, NS = 2, 16
NW = NC * NS
BPW = BN // NW
CH = 128
TOT_CH = E // CH
NCHUNK = 392
UNP = 50048
PERT = UNP // NS
HALF = F // 2


def _leaky(x):
    return jnp.where(x > 0, x, 0.01 * x)


def _item_mlp_body(x_ref, w1_ref, b1_ref, w2_ref, b2_ref, lo_ref, hi_ref):
    x = x_ref[...]
    h = _leaky(jnp.dot(x, w1_ref[...], preferred_element_type=jnp.float32)
               + b1_ref[...])
    o = (jnp.dot(h, w2_ref[...], preferred_element_type=jnp.float32)
         + b2_ref[...]) * 0.5
    lo_ref[...] = o[:, :HALF]
    hi_ref[...] = o[:, HALF:]


def _item_mlp(items, w1c, b1c, w2c, b2s):
    blk = 2000
    grid = IN_ // blk
    return pl.pallas_call(
        _item_mlp_body,
        grid=(grid,),
        in_specs=[
            pl.BlockSpec((blk, F), lambda i: (i, 0)),
            pl.BlockSpec((F, 4 * F), lambda i: (0, 0)),
            pl.BlockSpec((1, 4 * F), lambda i: (0, 0)),
            pl.BlockSpec((4 * F, F), lambda i: (0, 0)),
            pl.BlockSpec((1, F), lambda i: (0, 0)),
        ],
        out_specs=[
            pl.BlockSpec((blk, HALF), lambda i: (i, 0)),
            pl.BlockSpec((blk, HALF), lambda i: (i, 0)),
        ],
        out_shape=[
            jax.ShapeDtypeStruct((IN_, HALF), jnp.float32),
            jax.ShapeDtypeStruct((IN_, HALF), jnp.float32),
        ],
    )(items, w1c, b1c, w2c, b2s)


def _batch_body(ue_ref, fb_ref, ib_ref, rb_ref, w1_ref, b1_ref, w2_ref,
                b2_ref, wd1_ref, bd1_ref, wd2_ref, bd2_ref, out_ref):
    x = ue_ref[...]
    h = _leaky(jnp.dot(x, w1_ref[...], preferred_element_type=jnp.float32)
               + b1_ref[...])
    ub = (jnp.dot(h, w2_ref[...], preferred_element_type=jnp.float32)
          + b2_ref[...]) * 0.5
    hd = _leaky(jnp.dot(ub, wd1_ref[...], preferred_element_type=jnp.float32)
                + bd1_ref[...])
    logits = (jnp.dot(hd, wd2_ref[...], preferred_element_type=jnp.float32)
              + bd2_ref[...])
    col = lax.broadcasted_iota(jnp.int32, logits.shape, 1)
    neg = jnp.float32(-1e30)
    lse1 = jnp.log(jnp.sum(jnp.exp(jnp.where(col < 2, logits, neg)), -1))
    lse2 = jnp.log(jnp.sum(
        jnp.exp(jnp.where((col >= 2) & (col < 5), logits, neg)), -1))
    gender = fb_ref[...][:, 0:1]
    age = fb_ref[...][:, 1:2]
    ll1 = jnp.sum(jnp.where(col == gender, logits, 0.0), -1)
    ll2 = jnp.sum(jnp.where(col == 2 + age, logits, 0.0), -1)
    d_loss1 = jnp.mean(lse1 - ll1)
    d_loss2 = jnp.mean(lse2 - ll2)
    ib = ib_ref[...]
    pred = jnp.sum(ub * ib, -1)
    loss_part = jnp.mean((pred - rb_ref[...][:, 0]) ** 2)
    l2 = 0.001 * jnp.mean(jnp.sum(ub * ub + ib * ib, -1))
    lps = loss_part + l2
    ocol = lax.broadcasted_iota(jnp.int32, (1, 128), 1)
    out_ref[...] = (jnp.where(ocol == 0, d_loss1, 0.0)
                    + jnp.where(ocol == 1, d_loss2, 0.0)
                    + jnp.where(ocol == 2, lps, 0.0))


def _batch_losses(ue, fb, ib, rb2, w1c, b1c, w2c, b2s, wd1, bd1, wd2, bd2):
    return pl.pallas_call(
        _batch_body,
        in_specs=[pl.BlockSpec(a.shape, lambda: tuple(0 for _ in a.shape))
                  for a in (ue, fb, ib, rb2, w1c, b1c, w2c, b2s, wd1, bd1,
                            wd2, bd2)],
        out_specs=pl.BlockSpec((1, 128), lambda: (0, 0)),
        out_shape=jax.ShapeDtypeStruct((1, 128), jnp.float32),
    )(ue, fb, ib, rb2, w1c, b1c, w2c, b2s, wd1, bd1, wd2, bd2)


_LBLK = 3128


def _local_body(lo_ref, hi_ref, ft_ref, wd1_ref, bd1_ref, wd2_ref, bd2_ref,
                out_ref):
    i = pl.program_id(0)
    ng = pl.num_programs(0)
    x = jnp.concatenate([lo_ref[...], hi_ref[...]], axis=-1)
    hd = _leaky(jnp.dot(x, wd1_ref[...], preferred_element_type=jnp.float32)
                + bd1_ref[...])
    logits = (jnp.dot(hd, wd2_ref[...], preferred_element_type=jnp.float32)
              + bd2_ref[...])
    col = lax.broadcasted_iota(jnp.int32, logits.shape, 1)
    neg = jnp.float32(-1e30)
    lse1 = jnp.log(jnp.sum(jnp.exp(jnp.where(col < 2, logits, neg)), -1))
    lse2 = jnp.log(jnp.sum(
        jnp.exp(jnp.where((col >= 2) & (col < 5), logits, neg)), -1))
    gender = ft_ref[...][:, 0:1]
    age = ft_ref[...][:, 1:2]
    ll1 = jnp.sum(jnp.where(col == gender, logits, 0.0), -1)
    ll2 = jnp.sum(jnp.where(col == 2 + age, logits, 0.0), -1)
    ridx = i * _LBLK + lax.broadcasted_iota(jnp.int32, (logits.shape[0],), 0)
    valid = (ridx < UN).astype(jnp.float32)
    s1 = jnp.sum((lse1 - ll1) * valid)
    s2 = jnp.sum((lse2 - ll2) * valid)
    ocol = lax.broadcasted_iota(jnp.int32, (1, 128), 1)
    part = (jnp.where(ocol == 0, s1, 0.0) + jnp.where(ocol == 1, s2, 0.0))

    @pl.when(i == 0)
    def _():
        out_ref[...] = jnp.zeros_like(out_ref)

    out_ref[...] += part

    @pl.when(i == ng - 1)
    def _():
        out_ref[...] = out_ref[...] * (1.0 / UN)


def _local_losses(acc_lo, acc_hi, feats, wd1, bd1, wd2, bd2):
    blk = _LBLK
    grid = UNP // blk
    return pl.pallas_call(
        _local_body,
        grid=(grid,),
        in_specs=[
            pl.BlockSpec((blk, HALF), lambda i: (i, 0)),
            pl.BlockSpec((blk, HALF), lambda i: (i, 0)),
            pl.BlockSpec((blk, 16), lambda i: (i, 0)),
            pl.BlockSpec((F, 2 * F), lambda i: (0, 0)),
            pl.BlockSpec((1, 2 * F), lambda i: (0, 0)),
            pl.BlockSpec((2 * F, 128), lambda i: (0, 0)),
            pl.BlockSpec((1, 128), lambda i: (0, 0)),
        ],
        out_specs=pl.BlockSpec((1, 128), lambda i: (0, 0)),
        out_shape=jax.ShapeDtypeStruct((1, 128), jnp.float32),
    )(acc_lo, acc_hi, feats, wd1, bd1, wd2, bd2)


QUAD = 4


def _sc_body(adj_row, adj_col, adj_val, item_cat, users_emb,
             user_batch, item_batch, feats,
             out_lo, out_hi, out_ue, out_iblo, out_ibhi, out_fb,
             rows0, rows1, rows2, rows3,
             col0, col1, col2, col3, row0, row1, row2, row3,
             val0, val1, val2, val3,
             bidx_v, bidx2_v, brow_v, bfeat_v,
             acc_sh, sg, ss, se):
    cid = lax.axis_index("c")
    sid = lax.axis_index("s")
    wid = sid * NC + cid
    rows_bufs = (rows0, rows1, rows2, rows3)
    col_bufs = (col0, col1, col2, col3)
    row_bufs = (row0, row1, row2, row3)
    val_bufs = (val0, val1, val2, val3)

    abase = wid * BPW
    pltpu.sync_copy(user_batch.at[pl.ds(abase, BPW)], bidx_v)
    for p in range(BPW // 32):
        pltpu.async_copy(users_emb.at[bidx_v.at[pl.ds(p * 32, 32)]],
                         brow_v, sg.at[0]).wait()
        pltpu.sync_copy(brow_v, out_ue.at[pl.ds(abase + p * 32, 32)])
    for p in range(BPW // 64):
        pltpu.async_copy(feats.at[bidx_v.at[pl.ds(p * 64, 64)]],
                         bfeat_v, sg.at[0]).wait()
        pltpu.sync_copy(bfeat_v, out_fb.at[pl.ds(abase + p * 64, 64)])
    pltpu.sync_copy(item_batch.at[pl.ds(abase, BPW)], bidx_v)
    off16 = jnp.full((16,), IN_, jnp.int32)
    for g in range(BPW // 16):
        bidx2_v[pl.ds(g * 16, 16)] = bidx_v[pl.ds(g * 16, 16)] + off16
    pltpu.async_copy(item_cat.at[bidx_v], rows0, sg.at[0]).wait()
    pltpu.sync_copy(rows0, out_iblo.at[pl.ds(abase, BPW)])
    pltpu.async_copy(item_cat.at[bidx2_v], rows0, sg.at[0]).wait()
    pltpu.sync_copy(rows0, out_ibhi.at[pl.ds(abase, BPW)])

    zeros16 = jnp.zeros((16,), jnp.float32)
    for i in range(CH):
        rows0[i, pl.ds(0, 16)] = zeros16
        rows0[i, pl.ds(16, 16)] = zeros16
    for g in range(BPW // 16):
        bidx_v[pl.ds(g * 16, 16)] = jnp.zeros((16,), jnp.int32)
    zbase = sid * PERT
    nfull = PERT // CH
    rem = PERT - nfull * CH
    for z in range(nfull):
        pltpu.sync_copy(rows0, acc_sh.at[pl.ds(zbase + z * CH, CH)])
    pltpu.sync_copy(rows0.at[pl.ds(0, rem)],
                    acc_sh.at[pl.ds(zbase + nfull * CH, rem)])
    plsc.subcore_barrier()

    coff = jnp.broadcast_to(cid * IN_, (16,)).astype(jnp.int32)
    tchunk = sid * NCHUNK

    for b in range(1, QUAD):
        pltpu.sync_copy(acc_sh.at[pl.ds(zbase, CH)], rows_bufs[b])
    for b in range(QUAD):
        pltpu.async_copy(rows_bufs[b], acc_sh.at[bidx_v], ss.at[b], add=True)

    def quad(k, c):
        for b in range(QUAD):
            j = tchunk + k * QUAD + b
            jj = jnp.minimum(j, TOT_CH - 1) * CH
            pltpu.make_async_copy(rows_bufs[b], acc_sh.at[bidx_v],
                                  ss.at[b]).wait()
            pltpu.async_copy(adj_col.at[pl.ds(jj, CH)], col_bufs[b], se.at[b])
            pltpu.async_copy(adj_row.at[pl.ds(jj, CH)], row_bufs[b], se.at[b])
            pltpu.async_copy(adj_val.at[pl.ds(jj, CH)], val_bufs[b], se.at[b])
        for b in range(QUAD):
            j = tchunk + k * QUAD + b
            jj = jnp.minimum(j, TOT_CH - 1) * CH
            pltpu.make_async_copy(adj_col.at[pl.ds(jj, CH)], col_bufs[b],
                                  se.at[b]).wait()
            pltpu.make_async_copy(adj_row.at[pl.ds(jj, CH)], row_bufs[b],
                                  se.at[b]).wait()
            pltpu.make_async_copy(adj_val.at[pl.ds(jj, CH)], val_bufs[b],
                                  se.at[b]).wait()

            @pl.when(j >= TOT_CH)
            def _():
                for g in range(CH // 16):
                    val_bufs[b][pl.ds(g * 16, 16)] = jnp.zeros(
                        (16,), jnp.float32)

            for g in range(CH // 16):
                col_bufs[b][pl.ds(g * 16, 16)] = (
                    col_bufs[b][pl.ds(g * 16, 16)] + coff)
            pltpu.async_copy(item_cat.at[col_bufs[b]], rows_bufs[b], sg.at[b])
        for b in range(QUAD):
            pltpu.make_async_copy(item_cat.at[col_bufs[b]], rows_bufs[b],
                                  sg.at[b]).wait()
            rb = rows_bufs[b]
            vb = val_bufs[b]
            for g in range(CH // 16):
                val16 = vb[pl.ds(g * 16, 16)]
                for t in range(16):
                    e = g * 16 + t
                    vv = jnp.broadcast_to(
                        lax.slice(val16, (t,), (t + 1,)), (16,))
                    rb[e, pl.ds(0, 16)] = rb[e, pl.ds(0, 16)] * vv
                    rb[e, pl.ds(16, 16)] = rb[e, pl.ds(16, 16)] * vv
            pltpu.async_copy(rb, acc_sh.at[row_bufs[b]], ss.at[b], add=True)
        return c

    lax.fori_loop(0, NCHUNK // QUAD, quad, 0)
    for b in range(QUAD):
        pltpu.make_async_copy(rows_bufs[b], acc_sh.at[bidx_v], ss.at[b]).wait()
    plsc.subcore_barrier()

    dbase = sid * PERT

    @pl.when(cid == 0)
    def _():
        pltpu.sync_copy(acc_sh.at[pl.ds(dbase, PERT)],
                        out_lo.at[pl.ds(dbase, PERT)])

    @pl.when(cid == 1)
    def _():
        pltpu.sync_copy(acc_sh.at[pl.ds(dbase, PERT)],
                        out_hi.at[pl.ds(dbase, PERT)])


def _sc_run(adj_row, adj_col, adj_val, item_cat, users_emb,
            user_batch, item_batch, feats_p):
    mesh = plsc.VectorSubcoreMesh(core_axis_name="c", subcore_axis_name="s",
                                  num_cores=NC, num_subcores=NS)
    f = functools.partial(
        pl.kernel,
        out_type=(
            jax.ShapeDtypeStruct((UNP, HALF), jnp.float32),
            jax.ShapeDtypeStruct((UNP, HALF), jnp.float32),
            jax.ShapeDtypeStruct((BN, F), jnp.float32),
            jax.ShapeDtypeStruct((BN, HALF), jnp.float32),
            jax.ShapeDtypeStruct((BN, HALF), jnp.float32),
            jax.ShapeDtypeStruct((BN, 16), jnp.int32),
        ),
        mesh=mesh,
        scratch_types=[
            pltpu.VMEM((CH, HALF), jnp.float32),
            pltpu.VMEM((CH, HALF), jnp.float32),
            pltpu.VMEM((CH, HALF), jnp.float32),
            pltpu.VMEM((CH, HALF), jnp.float32),
            pltpu.VMEM((CH,), jnp.int32),
            pltpu.VMEM((CH,), jnp.int32),
            pltpu.VMEM((CH,), jnp.int32),
            pltpu.VMEM((CH,), jnp.int32),
            pltpu.VMEM((CH,), jnp.int32),
            pltpu.VMEM((CH,), jnp.int32),
            pltpu.VMEM((CH,), jnp.int32),
            pltpu.VMEM((CH,), jnp.int32),
            pltpu.VMEM((CH,), jnp.float32),
            pltpu.VMEM((CH,), jnp.float32),
            pltpu.VMEM((CH,), jnp.float32),
            pltpu.VMEM((CH,), jnp.float32),
            pltpu.VMEM((BPW,), jnp.int32),
            pltpu.VMEM((BPW,), jnp.int32),
            pltpu.VMEM((32, F), jnp.float32),
            pltpu.VMEM((64, 16), jnp.int32),
            pltpu.VMEM_SHARED((UNP, HALF), jnp.float32),
            pltpu.SemaphoreType.DMA((QUAD,)),
            pltpu.SemaphoreType.DMA((QUAD,)),
            pltpu.SemaphoreType.DMA((QUAD,)),
        ],
        compiler_params=pltpu.CompilerParams(use_tc_tiling_on_sc=False),
    )(_sc_body)
    return f(adj_row, adj_col, adj_val, item_cat, users_emb,
             user_batch, item_batch, feats_p)


def kernel(adj_row, adj_col, adj_val, user_batch, rating_batch, item_batch,
           flag_t, users_features, gcn_users_embedding0, gcn_items_embedding0,
           f1_w1, f1_b1, f1_w2, f1_b2, f2_w1, f2_b1, f2_w2, f2_b2,
           d1_w1, d1_b1, d1_w2, d1_b2, d2_w1, d2_b1, d2_w2, d2_b2):
    w1c = jnp.concatenate([f1_w1, f2_w1], axis=1)
    b1c = jnp.concatenate([f1_b1, f2_b1])[None, :]
    w2c = jnp.concatenate([f1_w2, f2_w2], axis=0)
    b2s = (f1_b2 + f2_b2)[None, :]
    wd1 = jnp.concatenate([d1_w1, d2_w1], axis=1)
    bd1 = jnp.concatenate([d1_b1, d2_b1])[None, :]
    wd2 = jnp.zeros((2 * F, 128), jnp.float32)
    wd2 = wd2.at[:F, 0:2].set(d1_w2).at[F:, 2:5].set(d2_w2)
    bd2 = jnp.zeros((128,), jnp.float32)
    bd2 = bd2.at[0:2].set(d1_b2).at[2:5].set(d2_b2)[None, :]

    feats_p = jnp.pad(users_features, ((0, UNP - UN), (0, 14)))

    item_lo, item_hi = _item_mlp(gcn_items_embedding0, w1c, b1c, w2c, b2s)
    item_cat = jnp.concatenate([item_lo, item_hi], axis=0)

    acc_lo, acc_hi, ue, ib_lo, ib_hi, fb = _sc_run(
        adj_row, adj_col, adj_val, item_cat,
        gcn_users_embedding0, user_batch, item_batch, feats_p)

    ib = jnp.concatenate([ib_lo, ib_hi], axis=1)
    rb2 = rating_batch[:, None]

    bout = _batch_losses(ue, fb, ib, rb2, w1c, b1c, w2c, b2s,
                         wd1, bd1, wd2, bd2)
    lout = _local_losses(acc_lo, acc_hi, feats_p, wd1, bd1, wd2, bd2)

    d_loss1 = bout[0, 0]
    d_loss2 = bout[0, 1]
    lps = bout[0, 2]
    d_loss1_local = lout[0, 0]
    d_loss2_local = lout[0, 1]

    d_loss = (d_loss1 * 2.0 + d_loss2) / 2.0
    d_loss_local = d_loss1_local * 2.0 + d_loss2_local
    d_loss_all = 10.0 * (d_loss + 0.5 * d_loss_local)
    g_loss_all = 0.1 * lps - d_loss_all
    g_d_loss_all = -d_loss_all
    return (d_loss_all, g_loss_all, g_d_loss_all)


# speedup vs baseline: 14.7281x; 1.0817x over previous
import functools

import jax
import jax.numpy as jnp
from jax import lax
from jax.experimental import pallas as pl
from jax.experimental.pallas import tpu as pltpu
from jax.experimental.pallas import tpu_sc as plsc

UN = 50000
IN_ = 20000
F = 64
BN = 4096
E = 800000

NC, NS = 2, 16
NW = NC * NS
BPW = BN // NW
CH = 128
TOT_CH = E // CH
NCHUNK = 392
UNP = 50048
PERT = UNP // NS
HALF = F // 2


def _leaky(x):
    return jnp.where(x > 0, x, 0.01 * x)


def _item_mlp_body(x_ref, w1_ref, b1_ref, w2_ref, b2_ref, lo_ref, hi_ref):
    x = x_ref[...]
    h = _leaky(jnp.dot(x, w1_ref[...], preferred_element_type=jnp.float32)
               + b1_ref[...])
    o = (jnp.dot(h, w2_ref[...], preferred_element_type=jnp.float32)
         + b2_ref[...]) * 0.5
    lo_ref[...] = o[:, :HALF]
    hi_ref[...] = o[:, HALF:]


def _item_mlp(items, w1c, b1c, w2c, b2s):
    blk = 2000
    grid = IN_ // blk
    return pl.pallas_call(
        _item_mlp_body,
        grid=(grid,),
        in_specs=[
            pl.BlockSpec((blk, F), lambda i: (i, 0)),
            pl.BlockSpec((F, 4 * F), lambda i: (0, 0)),
            pl.BlockSpec((1, 4 * F), lambda i: (0, 0)),
            pl.BlockSpec((4 * F, F), lambda i: (0, 0)),
            pl.BlockSpec((1, F), lambda i: (0, 0)),
        ],
        out_specs=[
            pl.BlockSpec((blk, HALF), lambda i: (i, 0)),
            pl.BlockSpec((blk, HALF), lambda i: (i, 0)),
        ],
        out_shape=[
            jax.ShapeDtypeStruct((IN_, HALF), jnp.float32),
            jax.ShapeDtypeStruct((IN_, HALF), jnp.float32),
        ],
    )(items, w1c, b1c, w2c, b2s)


def _batch_body(ue_ref, fb_ref, ib_ref, rb_ref, w1_ref, b1_ref, w2_ref,
                b2_ref, wd1_ref, bd1_ref, wd2_ref, bd2_ref, out_ref):
    x = ue_ref[...]
    h = _leaky(jnp.dot(x, w1_ref[...], preferred_element_type=jnp.float32)
               + b1_ref[...])
    ub = (jnp.dot(h, w2_ref[...], preferred_element_type=jnp.float32)
          + b2_ref[...]) * 0.5
    hd = _leaky(jnp.dot(ub, wd1_ref[...], preferred_element_type=jnp.float32)
                + bd1_ref[...])
    logits = (jnp.dot(hd, wd2_ref[...], preferred_element_type=jnp.float32)
              + bd2_ref[...])
    col = lax.broadcasted_iota(jnp.int32, logits.shape, 1)
    neg = jnp.float32(-1e30)
    lse1 = jnp.log(jnp.sum(jnp.exp(jnp.where(col < 2, logits, neg)), -1))
    lse2 = jnp.log(jnp.sum(
        jnp.exp(jnp.where((col >= 2) & (col < 5), logits, neg)), -1))
    gender = fb_ref[...][:, 0:1]
    age = fb_ref[...][:, 1:2]
    ll1 = jnp.sum(jnp.where(col == gender, logits, 0.0), -1)
    ll2 = jnp.sum(jnp.where(col == 2 + age, logits, 0.0), -1)
    d_loss1 = jnp.mean(lse1 - ll1)
    d_loss2 = jnp.mean(lse2 - ll2)
    ib = ib_ref[...]
    pred = jnp.sum(ub * ib, -1)
    loss_part = jnp.mean((pred - rb_ref[...][:, 0]) ** 2)
    l2 = 0.001 * jnp.mean(jnp.sum(ub * ub + ib * ib, -1))
    lps = loss_part + l2
    ocol = lax.broadcasted_iota(jnp.int32, (1, 128), 1)
    out_ref[...] = (jnp.where(ocol == 0, d_loss1, 0.0)
                    + jnp.where(ocol == 1, d_loss2, 0.0)
                    + jnp.where(ocol == 2, lps, 0.0))


def _batch_losses(ue, fb, ib, rb2, w1c, b1c, w2c, b2s, wd1, bd1, wd2, bd2):
    return pl.pallas_call(
        _batch_body,
        in_specs=[pl.BlockSpec(a.shape, lambda: tuple(0 for _ in a.shape))
                  for a in (ue, fb, ib, rb2, w1c, b1c, w2c, b2s, wd1, bd1,
                            wd2, bd2)],
        out_specs=pl.BlockSpec((1, 128), lambda: (0, 0)),
        out_shape=jax.ShapeDtypeStruct((1, 128), jnp.float32),
    )(ue, fb, ib, rb2, w1c, b1c, w2c, b2s, wd1, bd1, wd2, bd2)


_LBLK = 2944


def _local_body(lo_ref, hi_ref, ft_ref, wd1_ref, bd1_ref, wd2_ref, bd2_ref,
                out_ref):
    i = pl.program_id(0)
    ng = pl.num_programs(0)
    blk = _LBLK
    x = jnp.concatenate([lo_ref[...], hi_ref[...]], axis=-1)
    hd = _leaky(jnp.dot(x, wd1_ref[...], preferred_element_type=jnp.float32)
                + bd1_ref[...])
    logits = (jnp.dot(hd, wd2_ref[...], preferred_element_type=jnp.float32)
              + bd2_ref[...])
    lt = logits.T
    l0 = lax.slice(lt, (0, 0), (1, blk))
    l1 = lax.slice(lt, (1, 0), (2, blk))
    l2 = lax.slice(lt, (2, 0), (3, blk))
    l3 = lax.slice(lt, (3, 0), (4, blk))
    l4 = lax.slice(lt, (4, 0), (5, blk))
    lse1 = jnp.log(jnp.exp(l0) + jnp.exp(l1))
    lse2 = jnp.log(jnp.exp(l2) + jnp.exp(l3) + jnp.exp(l4))
    ftt = ft_ref[...].astype(jnp.float32).T
    g = lax.slice(ftt, (0, 0), (1, blk))
    a = lax.slice(ftt, (1, 0), (2, blk))
    ll1 = l0 * (1.0 - g) + l1 * g
    ll2 = (l2 * (a == 0).astype(jnp.float32)
           + l3 * (a == 1).astype(jnp.float32)
           + l4 * (a == 2).astype(jnp.float32))
    ridx = i * blk + lax.broadcasted_iota(jnp.int32, (1, blk), 1)
    valid = (ridx < UN).astype(jnp.float32)
    s1 = jnp.sum((lse1 - ll1) * valid)
    s2 = jnp.sum((lse2 - ll2) * valid)
    ocol = lax.broadcasted_iota(jnp.int32, (1, 128), 1)
    part = (jnp.where(ocol == 0, s1, 0.0) + jnp.where(ocol == 1, s2, 0.0))

    @pl.when(i == 0)
    def _():
        out_ref[...] = jnp.zeros_like(out_ref)

    out_ref[...] += part

    @pl.when(i == ng - 1)
    def _():
        out_ref[...] = out_ref[...] * (1.0 / UN)


def _local_losses(acc_lo, acc_hi, feats, wd1, bd1, wd2, bd2):
    blk = _LBLK
    grid = UNP // blk
    return pl.pallas_call(
        _local_body,
        grid=(grid,),
        in_specs=[
            pl.BlockSpec((blk, HALF), lambda i: (i, 0)),
            pl.BlockSpec((blk, HALF), lambda i: (i, 0)),
            pl.BlockSpec((blk, 16), lambda i: (i, 0)),
            pl.BlockSpec((F, 2 * F), lambda i: (0, 0)),
            pl.BlockSpec((1, 2 * F), lambda i: (0, 0)),
            pl.BlockSpec((2 * F, 128), lambda i: (0, 0)),
            pl.BlockSpec((1, 128), lambda i: (0, 0)),
        ],
        out_specs=pl.BlockSpec((1, 128), lambda i: (0, 0)),
        out_shape=jax.ShapeDtypeStruct((1, 128), jnp.float32),
    )(acc_lo, acc_hi, feats, wd1, bd1, wd2, bd2)


QUAD = 4


def _sc_body(adj_row, adj_col, adj_val, item_cat, users_emb,
             user_batch, item_batch, feats,
             out_lo, out_hi, out_ue, out_iblo, out_ibhi, out_fb,
             rows0, rows1, rows2, rows3,
             col0, col1, col2, col3, row0, row1, row2, row3,
             val0, val1, val2, val3,
             bidx_v, bidx2_v, brow_v, bfeat_v,
             acc_sh, sg, ss, se):
    cid = lax.axis_index("c")
    sid = lax.axis_index("s")
    wid = sid * NC + cid
    rows_bufs = (rows0, rows1, rows2, rows3)
    col_bufs = (col0, col1, col2, col3)
    row_bufs = (row0, row1, row2, row3)
    val_bufs = (val0, val1, val2, val3)

    abase = wid * BPW
    pltpu.sync_copy(user_batch.at[pl.ds(abase, BPW)], bidx_v)
    for p in range(BPW // 32):
        pltpu.async_copy(users_emb.at[bidx_v.at[pl.ds(p * 32, 32)]],
                         brow_v, sg.at[0]).wait()
        pltpu.sync_copy(brow_v, out_ue.at[pl.ds(abase + p * 32, 32)])
    for p in range(BPW // 64):
        pltpu.async_copy(feats.at[bidx_v.at[pl.ds(p * 64, 64)]],
                         bfeat_v, sg.at[0]).wait()
        pltpu.sync_copy(bfeat_v, out_fb.at[pl.ds(abase + p * 64, 64)])
    pltpu.sync_copy(item_batch.at[pl.ds(abase, BPW)], bidx_v)
    off16 = jnp.full((16,), IN_, jnp.int32)
    for g in range(BPW // 16):
        bidx2_v[pl.ds(g * 16, 16)] = bidx_v[pl.ds(g * 16, 16)] + off16
    pltpu.async_copy(item_cat.at[bidx_v], rows0, sg.at[0]).wait()
    pltpu.sync_copy(rows0, out_iblo.at[pl.ds(abase, BPW)])
    pltpu.async_copy(item_cat.at[bidx2_v], rows0, sg.at[0]).wait()
    pltpu.sync_copy(rows0, out_ibhi.at[pl.ds(abase, BPW)])

    zeros16 = jnp.zeros((16,), jnp.float32)
    for i in range(CH):
        rows0[i, pl.ds(0, 16)] = zeros16
        rows0[i, pl.ds(16, 16)] = zeros16
    for g in range(BPW // 16):
        bidx_v[pl.ds(g * 16, 16)] = jnp.zeros((16,), jnp.int32)
    zbase = sid * PERT
    nfull = PERT // CH
    rem = PERT - nfull * CH
    for z in range(nfull):
        pltpu.sync_copy(rows0, acc_sh.at[pl.ds(zbase + z * CH, CH)])
    pltpu.sync_copy(rows0.at[pl.ds(0, rem)],
                    acc_sh.at[pl.ds(zbase + nfull * CH, rem)])
    plsc.subcore_barrier()

    coff = jnp.broadcast_to(cid * IN_, (16,)).astype(jnp.int32)
    tchunk = sid * NCHUNK

    for b in range(1, QUAD):
        pltpu.sync_copy(acc_sh.at[pl.ds(zbase, CH)], rows_bufs[b])
    for b in range(QUAD):
        pltpu.async_copy(rows_bufs[b], acc_sh.at[bidx_v], ss.at[b], add=True)

    def quad(k, c):
        for b in range(QUAD):
            j = tchunk + k * QUAD + b
            jj = jnp.minimum(j, TOT_CH - 1) * CH
            pltpu.make_async_copy(rows_bufs[b], acc_sh.at[bidx_v],
                                  ss.at[b]).wait()
            pltpu.async_copy(adj_col.at[pl.ds(jj, CH)], col_bufs[b], se.at[b])
            pltpu.async_copy(adj_row.at[pl.ds(jj, CH)], row_bufs[b], se.at[b])
            pltpu.async_copy(adj_val.at[pl.ds(jj, CH)], val_bufs[b], se.at[b])
        for b in range(QUAD):
            j = tchunk + k * QUAD + b
            jj = jnp.minimum(j, TOT_CH - 1) * CH
            pltpu.make_async_copy(adj_col.at[pl.ds(jj, CH)], col_bufs[b],
                                  se.at[b]).wait()
            pltpu.make_async_copy(adj_row.at[pl.ds(jj, CH)], row_bufs[b],
                                  se.at[b]).wait()
            pltpu.make_async_copy(adj_val.at[pl.ds(jj, CH)], val_bufs[b],
                                  se.at[b]).wait()

            @pl.when(j >= TOT_CH)
            def _():
                for g in range(CH // 16):
                    val_bufs[b][pl.ds(g * 16, 16)] = jnp.zeros(
                        (16,), jnp.float32)

            for g in range(CH // 16):
                col_bufs[b][pl.ds(g * 16, 16)] = (
                    col_bufs[b][pl.ds(g * 16, 16)] + coff)
            pltpu.async_copy(item_cat.at[col_bufs[b]], rows_bufs[b], sg.at[b])
        for b in range(QUAD):
            pltpu.make_async_copy(item_cat.at[col_bufs[b]], rows_bufs[b],
                                  sg.at[b]).wait()
            rb = rows_bufs[b]
            vb = val_bufs[b]
            for g in range(CH // 16):
                val16 = vb[pl.ds(g * 16, 16)]
                for t in range(16):
                    e = g * 16 + t
                    vv = jnp.broadcast_to(
                        lax.slice(val16, (t,), (t + 1,)), (16,))
                    rb[e, pl.ds(0, 16)] = rb[e, pl.ds(0, 16)] * vv
                    rb[e, pl.ds(16, 16)] = rb[e, pl.ds(16, 16)] * vv
            pltpu.async_copy(rb, acc_sh.at[row_bufs[b]], ss.at[b], add=True)
        return c

    lax.fori_loop(0, NCHUNK // QUAD, quad, 0)
    for b in range(QUAD):
        pltpu.make_async_copy(rows_bufs[b], acc_sh.at[bidx_v], ss.at[b]).wait()
    plsc.subcore_barrier()

    dbase = sid * PERT

    @pl.when(cid == 0)
    def _():
        pltpu.sync_copy(acc_sh.at[pl.ds(dbase, PERT)],
                        out_lo.at[pl.ds(dbase, PERT)])

    @pl.when(cid == 1)
    def _():
        pltpu.sync_copy(acc_sh.at[pl.ds(dbase, PERT)],
                        out_hi.at[pl.ds(dbase, PERT)])


def _sc_run(adj_row, adj_col, adj_val, item_cat, users_emb,
            user_batch, item_batch, feats_p):
    mesh = plsc.VectorSubcoreMesh(core_axis_name="c", subcore_axis_name="s",
                                  num_cores=NC, num_subcores=NS)
    f = functools.partial(
        pl.kernel,
        out_type=(
            jax.ShapeDtypeStruct((UNP, HALF), jnp.float32),
            jax.ShapeDtypeStruct((UNP, HALF), jnp.float32),
            jax.ShapeDtypeStruct((BN, F), jnp.float32),
            jax.ShapeDtypeStruct((BN, HALF), jnp.float32),
            jax.ShapeDtypeStruct((BN, HALF), jnp.float32),
            jax.ShapeDtypeStruct((BN, 16), jnp.int32),
        ),
        mesh=mesh,
        scratch_types=[
            pltpu.VMEM((CH, HALF), jnp.float32),
            pltpu.VMEM((CH, HALF), jnp.float32),
            pltpu.VMEM((CH, HALF), jnp.float32),
            pltpu.VMEM((CH, HALF), jnp.float32),
            pltpu.VMEM((CH,), jnp.int32),
            pltpu.VMEM((CH,), jnp.int32),
            pltpu.VMEM((CH,), jnp.int32),
            pltpu.VMEM((CH,), jnp.int32),
            pltpu.VMEM((CH,), jnp.int32),
            pltpu.VMEM((CH,), jnp.int32),
            pltpu.VMEM((CH,), jnp.int32),
            pltpu.VMEM((CH,), jnp.int32),
            pltpu.VMEM((CH,), jnp.float32),
            pltpu.VMEM((CH,), jnp.float32),
            pltpu.VMEM((CH,), jnp.float32),
            pltpu.VMEM((CH,), jnp.float32),
            pltpu.VMEM((BPW,), jnp.int32),
            pltpu.VMEM((BPW,), jnp.int32),
            pltpu.VMEM((32, F), jnp.float32),
            pltpu.VMEM((64, 16), jnp.int32),
            pltpu.VMEM_SHARED((UNP, HALF), jnp.float32),
            pltpu.SemaphoreType.DMA((QUAD,)),
            pltpu.SemaphoreType.DMA((QUAD,)),
            pltpu.SemaphoreType.DMA((QUAD,)),
        ],
        compiler_params=pltpu.CompilerParams(use_tc_tiling_on_sc=False),
    )(_sc_body)
    return f(adj_row, adj_col, adj_val, item_cat, users_emb,
             user_batch, item_batch, feats_p)


def kernel(adj_row, adj_col, adj_val, user_batch, rating_batch, item_batch,
           flag_t, users_features, gcn_users_embedding0, gcn_items_embedding0,
           f1_w1, f1_b1, f1_w2, f1_b2, f2_w1, f2_b1, f2_w2, f2_b2,
           d1_w1, d1_b1, d1_w2, d1_b2, d2_w1, d2_b1, d2_w2, d2_b2):
    w1c = jnp.concatenate([f1_w1, f2_w1], axis=1)
    b1c = jnp.concatenate([f1_b1, f2_b1])[None, :]
    w2c = jnp.concatenate([f1_w2, f2_w2], axis=0)
    b2s = (f1_b2 + f2_b2)[None, :]
    wd1 = jnp.concatenate([d1_w1, d2_w1], axis=1)
    bd1 = jnp.concatenate([d1_b1, d2_b1])[None, :]
    wd2 = jnp.zeros((2 * F, 128), jnp.float32)
    wd2 = wd2.at[:F, 0:2].set(d1_w2).at[F:, 2:5].set(d2_w2)
    bd2 = jnp.zeros((128,), jnp.float32)
    bd2 = bd2.at[0:2].set(d1_b2).at[2:5].set(d2_b2)[None, :]

    feats_p = jnp.pad(users_features, ((0, UNP - UN), (0, 14)))

    item_lo, item_hi = _item_mlp(gcn_items_embedding0, w1c, b1c, w2c, b2s)
    item_cat = jnp.concatenate([item_lo, item_hi], axis=0)

    acc_lo, acc_hi, ue, ib_lo, ib_hi, fb = _sc_run(
        adj_row, adj_col, adj_val, item_cat,
        gcn_users_embedding0, user_batch, item_batch, feats_p)

    ib = jnp.concatenate([ib_lo, ib_hi], axis=1)
    rb2 = rating_batch[:, None]

    bout = _batch_losses(ue, fb, ib, rb2, w1c, b1c, w2c, b2s,
                         wd1, bd1, wd2, bd2)
    lout = _local_losses(acc_lo, acc_hi, feats_p, wd1, bd1, wd2, bd2)

    d_loss1 = bout[0, 0]
    d_loss2 = bout[0, 1]
    lps = bout[0, 2]
    d_loss1_local = lout[0, 0]
    d_loss2_local = lout[0, 1]

    d_loss = (d_loss1 * 2.0 + d_loss2) / 2.0
    d_loss_local = d_loss1_local * 2.0 + d_loss2_local
    d_loss_all = 10.0 * (d_loss + 0.5 * d_loss_local)
    g_loss_all = 0.1 * lps - d_loss_all
    g_d_loss_all = -d_loss_all
    return (d_loss_all, g_loss_all, g_d_loss_all)


# speedup vs baseline: 16.2134x; 1.1008x over previous
import functools

import jax
import jax.numpy as jnp
from jax import lax
from jax.experimental import pallas as pl
from jax.experimental.pallas import tpu as pltpu
from jax.experimental.pallas import tpu_sc as plsc

UN = 50000
IN_ = 20000
F = 64
BN = 4096
E = 800000

NC, NS = 2, 16
NW = NC * NS
BPW = BN // NW
CH = 128
TOT_CH = E // CH
NCHUNK = 392
UNP = 50048
PERT = UNP // NS
HALF = F // 2


def _leaky(x):
    return jnp.where(x > 0, x, 0.01 * x)


def _item_mlp_body(x_ref, w1_ref, b1_ref, w2_ref, b2_ref, lo_ref, hi_ref):
    x = x_ref[...]
    h = _leaky(jnp.dot(x, w1_ref[...], preferred_element_type=jnp.float32)
               + b1_ref[...])
    o = (jnp.dot(h, w2_ref[...], preferred_element_type=jnp.float32)
         + b2_ref[...]) * 0.5
    lo_ref[...] = o[:, :HALF]
    hi_ref[...] = o[:, HALF:]


def _item_mlp(items, w1c, b1c, w2c, b2s):
    blk = 2000
    grid = IN_ // blk
    return pl.pallas_call(
        _item_mlp_body,
        grid=(grid,),
        in_specs=[
            pl.BlockSpec((blk, F), lambda i: (i, 0)),
            pl.BlockSpec((F, 4 * F), lambda i: (0, 0)),
            pl.BlockSpec((1, 4 * F), lambda i: (0, 0)),
            pl.BlockSpec((4 * F, F), lambda i: (0, 0)),
            pl.BlockSpec((1, F), lambda i: (0, 0)),
        ],
        out_specs=[
            pl.BlockSpec((blk, HALF), lambda i: (i, 0)),
            pl.BlockSpec((blk, HALF), lambda i: (i, 0)),
        ],
        out_shape=[
            jax.ShapeDtypeStruct((IN_, HALF), jnp.float32),
            jax.ShapeDtypeStruct((IN_, HALF), jnp.float32),
        ],
    )(items, w1c, b1c, w2c, b2s)


def _batch_body(ue_ref, fb_ref, ib_ref, rb_ref, w1_ref, b1_ref, w2_ref,
                b2_ref, wd1_ref, bd1_ref, wd2_ref, bd2_ref, out_ref):
    x = ue_ref[...]
    h = _leaky(jnp.dot(x, w1_ref[...], preferred_element_type=jnp.float32)
               + b1_ref[...])
    ub = (jnp.dot(h, w2_ref[...], preferred_element_type=jnp.float32)
          + b2_ref[...]) * 0.5
    hd = _leaky(jnp.dot(ub, wd1_ref[...], preferred_element_type=jnp.float32)
                + bd1_ref[...])
    logits = (jnp.dot(hd, wd2_ref[...], preferred_element_type=jnp.float32)
              + bd2_ref[...])
    col = lax.broadcasted_iota(jnp.int32, logits.shape, 1)
    neg = jnp.float32(-1e30)
    lse1 = jnp.log(jnp.sum(jnp.exp(jnp.where(col < 2, logits, neg)), -1))
    lse2 = jnp.log(jnp.sum(
        jnp.exp(jnp.where((col >= 2) & (col < 5), logits, neg)), -1))
    gender = fb_ref[...][:, 0:1]
    age = fb_ref[...][:, 1:2]
    ll1 = jnp.sum(jnp.where(col == gender, logits, 0.0), -1)
    ll2 = jnp.sum(jnp.where(col == 2 + age, logits, 0.0), -1)
    d_loss1 = jnp.mean(lse1 - ll1)
    d_loss2 = jnp.mean(lse2 - ll2)
    ib = ib_ref[...]
    pred = jnp.sum(ub * ib, -1)
    loss_part = jnp.mean((pred - rb_ref[...][:, 0]) ** 2)
    l2 = 0.001 * jnp.mean(jnp.sum(ub * ub + ib * ib, -1))
    lps = loss_part + l2
    ocol = lax.broadcasted_iota(jnp.int32, (1, 128), 1)
    out_ref[...] = (jnp.where(ocol == 0, d_loss1, 0.0)
                    + jnp.where(ocol == 1, d_loss2, 0.0)
                    + jnp.where(ocol == 2, lps, 0.0))


def _batch_losses(ue, fb, ib, rb2, w1c, b1c, w2c, b2s, wd1, bd1, wd2, bd2):
    return pl.pallas_call(
        _batch_body,
        in_specs=[pl.BlockSpec(a.shape, lambda: tuple(0 for _ in a.shape))
                  for a in (ue, fb, ib, rb2, w1c, b1c, w2c, b2s, wd1, bd1,
                            wd2, bd2)],
        out_specs=pl.BlockSpec((1, 128), lambda: (0, 0)),
        out_shape=jax.ShapeDtypeStruct((1, 128), jnp.float32),
    )(ue, fb, ib, rb2, w1c, b1c, w2c, b2s, wd1, bd1, wd2, bd2)


_LBLK = 2944


def _local_body(lo_ref, hi_ref, ft_ref, wd1_ref, bd1_ref, wd2_ref, bd2_ref,
                out_ref):
    i = pl.program_id(0)
    ng = pl.num_programs(0)
    blk = _LBLK
    x = jnp.concatenate([lo_ref[...], hi_ref[...]], axis=-1)
    hd = _leaky(jnp.dot(x, wd1_ref[...], preferred_element_type=jnp.float32)
                + bd1_ref[...])
    logits = (jnp.dot(hd, wd2_ref[...], preferred_element_type=jnp.float32)
              + bd2_ref[...])
    lt = logits.T
    l0 = lax.slice(lt, (0, 0), (1, blk))
    l1 = lax.slice(lt, (1, 0), (2, blk))
    l2 = lax.slice(lt, (2, 0), (3, blk))
    l3 = lax.slice(lt, (3, 0), (4, blk))
    l4 = lax.slice(lt, (4, 0), (5, blk))
    lse1 = jnp.log(jnp.exp(l0) + jnp.exp(l1))
    lse2 = jnp.log(jnp.exp(l2) + jnp.exp(l3) + jnp.exp(l4))
    ftt = ft_ref[...].astype(jnp.float32).T
    g = lax.slice(ftt, (0, 0), (1, blk))
    a = lax.slice(ftt, (1, 0), (2, blk))
    ll1 = l0 * (1.0 - g) + l1 * g
    ll2 = (l2 * (a == 0).astype(jnp.float32)
           + l3 * (a == 1).astype(jnp.float32)
           + l4 * (a == 2).astype(jnp.float32))
    ridx = i * blk + lax.broadcasted_iota(jnp.int32, (1, blk), 1)
    valid = (ridx < UN).astype(jnp.float32)
    s1 = jnp.sum((lse1 - ll1) * valid)
    s2 = jnp.sum((lse2 - ll2) * valid)
    ocol = lax.broadcasted_iota(jnp.int32, (1, 128), 1)
    part = (jnp.where(ocol == 0, s1, 0.0) + jnp.where(ocol == 1, s2, 0.0))

    @pl.when(i == 0)
    def _():
        out_ref[...] = jnp.zeros_like(out_ref)

    out_ref[...] += part

    @pl.when(i == ng - 1)
    def _():
        out_ref[...] = out_ref[...] * (1.0 / UN)


def _local_losses(acc_lo, acc_hi, feats, wd1, bd1, wd2, bd2):
    blk = _LBLK
    grid = UNP // blk
    return pl.pallas_call(
        _local_body,
        grid=(grid,),
        in_specs=[
            pl.BlockSpec((blk, HALF), lambda i: (i, 0)),
            pl.BlockSpec((blk, HALF), lambda i: (i, 0)),
            pl.BlockSpec((blk, 16), lambda i: (i, 0)),
            pl.BlockSpec((F, 2 * F), lambda i: (0, 0)),
            pl.BlockSpec((1, 2 * F), lambda i: (0, 0)),
            pl.BlockSpec((2 * F, 128), lambda i: (0, 0)),
            pl.BlockSpec((1, 128), lambda i: (0, 0)),
        ],
        out_specs=pl.BlockSpec((1, 128), lambda i: (0, 0)),
        out_shape=jax.ShapeDtypeStruct((1, 128), jnp.float32),
    )(acc_lo, acc_hi, feats, wd1, bd1, wd2, bd2)


QUAD = 4


def _sc_edges_body(adj_row, adj_col, adj_val, item_cat,
                   out_lo, out_hi,
                   rows0, rows1, rows2, rows3,
                   col0, col1, col2, col3, row0, row1, row2, row3,
                   val0, val1, val2, val3,
                   bidx_v, acc_sh, sg, ss, se):
    cid = lax.axis_index("c")
    sid = lax.axis_index("s")
    rows_bufs = (rows0, rows1, rows2, rows3)
    col_bufs = (col0, col1, col2, col3)
    row_bufs = (row0, row1, row2, row3)
    val_bufs = (val0, val1, val2, val3)

    zeros16 = jnp.zeros((16,), jnp.float32)
    for i in range(CH):
        rows0[i, pl.ds(0, 16)] = zeros16
        rows0[i, pl.ds(16, 16)] = zeros16
    for g in range(BPW // 16):
        bidx_v[pl.ds(g * 16, 16)] = jnp.zeros((16,), jnp.int32)
    zbase = sid * PERT
    nfull = PERT // CH
    rem = PERT - nfull * CH
    for z in range(nfull):
        pltpu.sync_copy(rows0, acc_sh.at[pl.ds(zbase + z * CH, CH)])
    pltpu.sync_copy(rows0.at[pl.ds(0, rem)],
                    acc_sh.at[pl.ds(zbase + nfull * CH, rem)])
    plsc.subcore_barrier()

    coff = jnp.broadcast_to(cid * IN_, (16,)).astype(jnp.int32)
    tchunk = sid * NCHUNK

    for b in range(1, QUAD):
        pltpu.sync_copy(acc_sh.at[pl.ds(zbase, CH)], rows_bufs[b])
    for b in range(QUAD):
        pltpu.async_copy(rows_bufs[b], acc_sh.at[bidx_v], ss.at[b], add=True)

    def quad(k, c):
        for b in range(QUAD):
            j = tchunk + k * QUAD + b
            jj = jnp.minimum(j, TOT_CH - 1) * CH
            pltpu.make_async_copy(rows_bufs[b], acc_sh.at[bidx_v],
                                  ss.at[b]).wait()
            pltpu.async_copy(adj_col.at[pl.ds(jj, CH)], col_bufs[b], se.at[b])
            pltpu.async_copy(adj_row.at[pl.ds(jj, CH)], row_bufs[b], se.at[b])
            pltpu.async_copy(adj_val.at[pl.ds(jj, CH)], val_bufs[b], se.at[b])
        for b in range(QUAD):
            j = tchunk + k * QUAD + b
            jj = jnp.minimum(j, TOT_CH - 1) * CH
            pltpu.make_async_copy(adj_col.at[pl.ds(jj, CH)], col_bufs[b],
                                  se.at[b]).wait()
            pltpu.make_async_copy(adj_row.at[pl.ds(jj, CH)], row_bufs[b],
                                  se.at[b]).wait()
            pltpu.make_async_copy(adj_val.at[pl.ds(jj, CH)], val_bufs[b],
                                  se.at[b]).wait()

            @pl.when(j >= TOT_CH)
            def _():
                for g in range(CH // 16):
                    val_bufs[b][pl.ds(g * 16, 16)] = jnp.zeros(
                        (16,), jnp.float32)

            for g in range(CH // 16):
                col_bufs[b][pl.ds(g * 16, 16)] = (
                    col_bufs[b][pl.ds(g * 16, 16)] + coff)
            pltpu.async_copy(item_cat.at[col_bufs[b]], rows_bufs[b], sg.at[b])
        for b in range(QUAD):
            pltpu.make_async_copy(item_cat.at[col_bufs[b]], rows_bufs[b],
                                  sg.at[b]).wait()
            rb = rows_bufs[b]
            vb = val_bufs[b]
            for g in range(CH // 16):
                val16 = vb[pl.ds(g * 16, 16)]
                for t in range(16):
                    e = g * 16 + t
                    vv = jnp.broadcast_to(
                        lax.slice(val16, (t,), (t + 1,)), (16,))
                    rb[e, pl.ds(0, 16)] = rb[e, pl.ds(0, 16)] * vv
                    rb[e, pl.ds(16, 16)] = rb[e, pl.ds(16, 16)] * vv
            pltpu.async_copy(rb, acc_sh.at[row_bufs[b]], ss.at[b], add=True)
        return c

    lax.fori_loop(0, NCHUNK // QUAD, quad, 0)
    for b in range(QUAD):
        pltpu.make_async_copy(rows_bufs[b], acc_sh.at[bidx_v], ss.at[b]).wait()
    plsc.subcore_barrier()

    dbase = sid * PERT

    @pl.when(cid == 0)
    def _():
        pltpu.sync_copy(acc_sh.at[pl.ds(dbase, PERT)],
                        out_lo.at[pl.ds(dbase, PERT)])

    @pl.when(cid == 1)
    def _():
        pltpu.sync_copy(acc_sh.at[pl.ds(dbase, PERT)],
                        out_hi.at[pl.ds(dbase, PERT)])


def _sc_batch_body(item_cat, users_emb, user_batch, item_batch, feats, dep,
                   out_ue, out_iblo, out_ibhi, out_fb,
                   rows0, bidx_v, bidx2_v, brow_v, bfeat_v, sg):
    cid = lax.axis_index("c")
    sid = lax.axis_index("s")
    wid = sid * NC + cid

    abase = wid * BPW
    pltpu.sync_copy(user_batch.at[pl.ds(abase, BPW)], bidx_v)
    for p in range(BPW // 32):
        pltpu.async_copy(users_emb.at[bidx_v.at[pl.ds(p * 32, 32)]],
                         brow_v, sg.at[0]).wait()
        pltpu.sync_copy(brow_v, out_ue.at[pl.ds(abase + p * 32, 32)])
    for p in range(BPW // 64):
        pltpu.async_copy(feats.at[bidx_v.at[pl.ds(p * 64, 64)]],
                         bfeat_v, sg.at[0]).wait()
        pltpu.sync_copy(bfeat_v, out_fb.at[pl.ds(abase + p * 64, 64)])
    pltpu.sync_copy(item_batch.at[pl.ds(abase, BPW)], bidx_v)
    off16 = jnp.full((16,), IN_, jnp.int32)
    for g in range(BPW // 16):
        bidx2_v[pl.ds(g * 16, 16)] = bidx_v[pl.ds(g * 16, 16)] + off16
    pltpu.async_copy(item_cat.at[bidx_v], rows0, sg.at[0]).wait()
    pltpu.sync_copy(rows0, out_iblo.at[pl.ds(abase, BPW)])
    pltpu.async_copy(item_cat.at[bidx2_v], rows0, sg.at[0]).wait()
    pltpu.sync_copy(rows0, out_ibhi.at[pl.ds(abase, BPW)])


def _sc_mesh():
    return plsc.VectorSubcoreMesh(core_axis_name="c", subcore_axis_name="s",
                                  num_cores=NC, num_subcores=NS)


def _sc_edges(adj_row, adj_col, adj_val, item_cat):
    f = functools.partial(
        pl.kernel,
        out_type=(
            jax.ShapeDtypeStruct((UNP, HALF), jnp.float32),
            jax.ShapeDtypeStruct((UNP, HALF), jnp.float32),
        ),
        mesh=_sc_mesh(),
        scratch_types=[
            pltpu.VMEM((CH, HALF), jnp.float32),
            pltpu.VMEM((CH, HALF), jnp.float32),
            pltpu.VMEM((CH, HALF), jnp.float32),
            pltpu.VMEM((CH, HALF), jnp.float32),
            pltpu.VMEM((CH,), jnp.int32),
            pltpu.VMEM((CH,), jnp.int32),
            pltpu.VMEM((CH,), jnp.int32),
            pltpu.VMEM((CH,), jnp.int32),
            pltpu.VMEM((CH,), jnp.int32),
            pltpu.VMEM((CH,), jnp.int32),
            pltpu.VMEM((CH,), jnp.int32),
            pltpu.VMEM((CH,), jnp.int32),
            pltpu.VMEM((CH,), jnp.float32),
            pltpu.VMEM((CH,), jnp.float32),
            pltpu.VMEM((CH,), jnp.float32),
            pltpu.VMEM((CH,), jnp.float32),
            pltpu.VMEM((CH,), jnp.int32),
            pltpu.VMEM_SHARED((UNP, HALF), jnp.float32),
            pltpu.SemaphoreType.DMA((QUAD,)),
            pltpu.SemaphoreType.DMA((QUAD,)),
            pltpu.SemaphoreType.DMA((QUAD,)),
        ],
        compiler_params=pltpu.CompilerParams(use_tc_tiling_on_sc=False),
    )(_sc_edges_body)
    return f(adj_row, adj_col, adj_val, item_cat)


def _sc_batch(item_cat, users_emb, user_batch, item_batch, feats_p, dep):
    f = functools.partial(
        pl.kernel,
        out_type=(
            jax.ShapeDtypeStruct((BN, F), jnp.float32),
            jax.ShapeDtypeStruct((BN, HALF), jnp.float32),
            jax.ShapeDtypeStruct((BN, HALF), jnp.float32),
            jax.ShapeDtypeStruct((BN, 16), jnp.int32),
        ),
        mesh=_sc_mesh(),
        scratch_types=[
            pltpu.VMEM((CH, HALF), jnp.float32),
            pltpu.VMEM((BPW,), jnp.int32),
            pltpu.VMEM((BPW,), jnp.int32),
            pltpu.VMEM((32, F), jnp.float32),
            pltpu.VMEM((64, 16), jnp.int32),
            pltpu.SemaphoreType.DMA((1,)),
        ],
        compiler_params=pltpu.CompilerParams(use_tc_tiling_on_sc=False),
    )(_sc_batch_body)
    return f(item_cat, users_emb, user_batch, item_batch, feats_p, dep)


def kernel(adj_row, adj_col, adj_val, user_batch, rating_batch, item_batch,
           flag_t, users_features, gcn_users_embedding0, gcn_items_embedding0,
           f1_w1, f1_b1, f1_w2, f1_b2, f2_w1, f2_b1, f2_w2, f2_b2,
           d1_w1, d1_b1, d1_w2, d1_b2, d2_w1, d2_b1, d2_w2, d2_b2):
    w1c = jnp.concatenate([f1_w1, f2_w1], axis=1)
    b1c = jnp.concatenate([f1_b1, f2_b1])[None, :]
    w2c = jnp.concatenate([f1_w2, f2_w2], axis=0)
    b2s = (f1_b2 + f2_b2)[None, :]
    wd1 = jnp.concatenate([d1_w1, d2_w1], axis=1)
    bd1 = jnp.concatenate([d1_b1, d2_b1])[None, :]
    wd2 = jnp.zeros((2 * F, 128), jnp.float32)
    wd2 = wd2.at[:F, 0:2].set(d1_w2).at[F:, 2:5].set(d2_w2)
    bd2 = jnp.zeros((128,), jnp.float32)
    bd2 = bd2.at[0:2].set(d1_b2).at[2:5].set(d2_b2)[None, :]

    feats_p = jnp.pad(users_features, ((0, UNP - UN), (0, 14)))

    item_lo, item_hi = _item_mlp(gcn_items_embedding0, w1c, b1c, w2c, b2s)
    item_cat = jnp.concatenate([item_lo, item_hi], axis=0)

    acc_lo, acc_hi = _sc_edges(adj_row, adj_col, adj_val, item_cat)
    ue, ib_lo, ib_hi, fb = _sc_batch(
        item_cat, gcn_users_embedding0, user_batch, item_batch, feats_p,
        acc_lo)

    ib = jnp.concatenate([ib_lo, ib_hi], axis=1)
    rb2 = rating_batch[:, None]

    bout = _batch_losses(ue, fb, ib, rb2, w1c, b1c, w2c, b2s,
                         wd1, bd1, wd2, bd2)
    lout = _local_losses(acc_lo, acc_hi, feats_p, wd1, bd1, wd2, bd2)

    d_loss1 = bout[0, 0]
    d_loss2 = bout[0, 1]
    lps = bout[0, 2]
    d_loss1_local = lout[0, 0]
    d_loss2_local = lout[0, 1]

    d_loss = (d_loss1 * 2.0 + d_loss2) / 2.0
    d_loss_local = d_loss1_local * 2.0 + d_loss2_local
    d_loss_all = 10.0 * (d_loss + 0.5 * d_loss_local)
    g_loss_all = 0.1 * lps - d_loss_all
    g_d_loss_all = -d_loss_all
    return (d_loss_all, g_loss_all, g_d_loss_all)


# speedup vs baseline: 17.6320x; 1.0875x over previous
import functools

import jax
import jax.numpy as jnp
from jax import lax
from jax.experimental import pallas as pl
from jax.experimental.pallas import tpu as pltpu
from jax.experimental.pallas import tpu_sc as plsc

UN = 50000
IN_ = 20000
F = 64
BN = 4096
E = 800000

NC, NS = 2, 16
NW = NC * NS
BPW = BN // NW
CH = 128
TOT_CH = E // CH
NCHUNK = 392
UNP = 50048
PERT = UNP // NS
HALF = F // 2


def _leaky(x):
    return jnp.where(x > 0, x, 0.01 * x)


def _item_mlp_body(x_ref, w1_ref, b1_ref, w2_ref, b2_ref, lo_ref, hi_ref):
    x = x_ref[...]
    h = _leaky(jnp.dot(x, w1_ref[...], preferred_element_type=jnp.float32)
               + b1_ref[...])
    o = (jnp.dot(h, w2_ref[...], preferred_element_type=jnp.float32)
         + b2_ref[...]) * 0.5
    lo_ref[...] = o[:, :HALF]
    hi_ref[...] = o[:, HALF:]


def _item_mlp(items, w1c, b1c, w2c, b2s):
    blk = 2000
    grid = IN_ // blk
    return pl.pallas_call(
        _item_mlp_body,
        grid=(grid,),
        in_specs=[
            pl.BlockSpec((blk, F), lambda i: (i, 0)),
            pl.BlockSpec((F, 4 * F), lambda i: (0, 0)),
            pl.BlockSpec((1, 4 * F), lambda i: (0, 0)),
            pl.BlockSpec((4 * F, F), lambda i: (0, 0)),
            pl.BlockSpec((1, F), lambda i: (0, 0)),
        ],
        out_specs=[
            pl.BlockSpec((blk, HALF), lambda i: (i, 0)),
            pl.BlockSpec((blk, HALF), lambda i: (i, 0)),
        ],
        out_shape=[
            jax.ShapeDtypeStruct((IN_, HALF), jnp.float32),
            jax.ShapeDtypeStruct((IN_, HALF), jnp.float32),
        ],
    )(items, w1c, b1c, w2c, b2s)


def _batch_body(ue_ref, fb_ref, ib_ref, rb_ref, w1_ref, b1_ref, w2_ref,
                b2_ref, wd1_ref, bd1_ref, wd2_ref, bd2_ref, out_ref):
    x = ue_ref[...]
    h = _leaky(jnp.dot(x, w1_ref[...], preferred_element_type=jnp.float32)
               + b1_ref[...])
    ub = (jnp.dot(h, w2_ref[...], preferred_element_type=jnp.float32)
          + b2_ref[...]) * 0.5
    hd = _leaky(jnp.dot(ub, wd1_ref[...], preferred_element_type=jnp.float32)
                + bd1_ref[...])
    logits = (jnp.dot(hd, wd2_ref[...], preferred_element_type=jnp.float32)
              + bd2_ref[...])
    col = lax.broadcasted_iota(jnp.int32, logits.shape, 1)
    neg = jnp.float32(-1e30)
    lse1 = jnp.log(jnp.sum(jnp.exp(jnp.where(col < 2, logits, neg)), -1))
    lse2 = jnp.log(jnp.sum(
        jnp.exp(jnp.where((col >= 2) & (col < 5), logits, neg)), -1))
    gender = fb_ref[...][:, 0:1]
    age = fb_ref[...][:, 1:2]
    ll1 = jnp.sum(jnp.where(col == gender, logits, 0.0), -1)
    ll2 = jnp.sum(jnp.where(col == 2 + age, logits, 0.0), -1)
    d_loss1 = jnp.mean(lse1 - ll1)
    d_loss2 = jnp.mean(lse2 - ll2)
    ib = ib_ref[...]
    pred = jnp.sum(ub * ib, -1)
    loss_part = jnp.mean((pred - rb_ref[...][:, 0]) ** 2)
    l2 = 0.001 * jnp.mean(jnp.sum(ub * ub + ib * ib, -1))
    lps = loss_part + l2
    ocol = lax.broadcasted_iota(jnp.int32, (1, 128), 1)
    out_ref[...] = (jnp.where(ocol == 0, d_loss1, 0.0)
                    + jnp.where(ocol == 1, d_loss2, 0.0)
                    + jnp.where(ocol == 2, lps, 0.0))


def _batch_losses(ue, fb, ib, rb2, w1c, b1c, w2c, b2s, wd1, bd1, wd2, bd2):
    return pl.pallas_call(
        _batch_body,
        in_specs=[pl.BlockSpec(a.shape, lambda: tuple(0 for _ in a.shape))
                  for a in (ue, fb, ib, rb2, w1c, b1c, w2c, b2s, wd1, bd1,
                            wd2, bd2)],
        out_specs=pl.BlockSpec((1, 128), lambda: (0, 0)),
        out_shape=jax.ShapeDtypeStruct((1, 128), jnp.float32),
    )(ue, fb, ib, rb2, w1c, b1c, w2c, b2s, wd1, bd1, wd2, bd2)


_LBLK = 2944


def _local_body(acc_ref, ft_ref, wd1_ref, bd1_ref, wd2_ref, bd2_ref,
                out_ref):
    i = pl.program_id(0)
    ng = pl.num_programs(0)
    blk = _LBLK
    x = lax.slice(acc_ref[...], (0, 0), (blk, F))
    hd = _leaky(jnp.dot(x, wd1_ref[...], preferred_element_type=jnp.float32)
                + bd1_ref[...])
    logits = (jnp.dot(hd, wd2_ref[...], preferred_element_type=jnp.float32)
              + bd2_ref[...])
    lt = logits.T
    l0 = lax.slice(lt, (0, 0), (1, blk))
    l1 = lax.slice(lt, (1, 0), (2, blk))
    l2 = lax.slice(lt, (2, 0), (3, blk))
    l3 = lax.slice(lt, (3, 0), (4, blk))
    l4 = lax.slice(lt, (4, 0), (5, blk))
    lse1 = jnp.log(jnp.exp(l0) + jnp.exp(l1))
    lse2 = jnp.log(jnp.exp(l2) + jnp.exp(l3) + jnp.exp(l4))
    ftt = ft_ref[...].astype(jnp.float32).T
    g = lax.slice(ftt, (0, 0), (1, blk))
    a = lax.slice(ftt, (1, 0), (2, blk))
    ll1 = l0 * (1.0 - g) + l1 * g
    ll2 = (l2 * (a == 0).astype(jnp.float32)
           + l3 * (a == 1).astype(jnp.float32)
           + l4 * (a == 2).astype(jnp.float32))
    ridx = i * blk + lax.broadcasted_iota(jnp.int32, (1, blk), 1)
    valid = (ridx < UN).astype(jnp.float32)
    s1 = jnp.sum((lse1 - ll1) * valid)
    s2 = jnp.sum((lse2 - ll2) * valid)
    ocol = lax.broadcasted_iota(jnp.int32, (1, 128), 1)
    part = (jnp.where(ocol == 0, s1, 0.0) + jnp.where(ocol == 1, s2, 0.0))

    @pl.when(i == 0)
    def _():
        out_ref[...] = jnp.zeros_like(out_ref)

    out_ref[...] += part

    @pl.when(i == ng - 1)
    def _():
        out_ref[...] = out_ref[...] * (1.0 / UN)


def _local_losses(acc, feats, wd1, bd1, wd2, bd2):
    blk = _LBLK
    grid = UNP // blk
    return pl.pallas_call(
        _local_body,
        grid=(grid,),
        in_specs=[
            pl.BlockSpec((blk, 128), lambda i: (i, 0)),
            pl.BlockSpec((blk, 16), lambda i: (i, 0)),
            pl.BlockSpec((F, 2 * F), lambda i: (0, 0)),
            pl.BlockSpec((1, 2 * F), lambda i: (0, 0)),
            pl.BlockSpec((2 * F, 128), lambda i: (0, 0)),
            pl.BlockSpec((1, 128), lambda i: (0, 0)),
        ],
        out_specs=pl.BlockSpec((1, 128), lambda i: (0, 0)),
        out_shape=jax.ShapeDtypeStruct((1, 128), jnp.float32),
    )(acc, feats, wd1, bd1, wd2, bd2)


QUAD = 4


def _sc_edges_body(adj_row, adj_col, adj_val, item_cat,
                   out_acc,
                   rows0, rows1, rows2, rows3,
                   col0, col1, col2, col3, row0, row1, row2, row3,
                   val0, val1, val2, val3,
                   bidx_v, acc_sh, sg, ss, se):
    cid = lax.axis_index("c")
    sid = lax.axis_index("s")
    rows_bufs = (rows0, rows1, rows2, rows3)
    col_bufs = (col0, col1, col2, col3)
    row_bufs = (row0, row1, row2, row3)
    val_bufs = (val0, val1, val2, val3)

    zeros16 = jnp.zeros((16,), jnp.float32)
    for i in range(CH):
        rows0[i, pl.ds(0, 16)] = zeros16
        rows0[i, pl.ds(16, 16)] = zeros16
    for g in range(BPW // 16):
        bidx_v[pl.ds(g * 16, 16)] = jnp.zeros((16,), jnp.int32)
    zbase = sid * PERT
    nfull = PERT // CH
    rem = PERT - nfull * CH
    for z in range(nfull):
        pltpu.sync_copy(rows0, acc_sh.at[pl.ds(zbase + z * CH, CH)])
    pltpu.sync_copy(rows0.at[pl.ds(0, rem)],
                    acc_sh.at[pl.ds(zbase + nfull * CH, rem)])
    plsc.subcore_barrier()

    coff = jnp.broadcast_to(cid * IN_, (16,)).astype(jnp.int32)
    tchunk = sid * NCHUNK

    for b in range(1, QUAD):
        pltpu.sync_copy(acc_sh.at[pl.ds(zbase, CH)], rows_bufs[b])
    for b in range(QUAD):
        pltpu.async_copy(rows_bufs[b], acc_sh.at[bidx_v], ss.at[b], add=True)

    def quad(k, c):
        for b in range(QUAD):
            j = tchunk + k * QUAD + b
            jj = jnp.minimum(j, TOT_CH - 1) * CH
            pltpu.make_async_copy(rows_bufs[b], acc_sh.at[bidx_v],
                                  ss.at[b]).wait()
            pltpu.async_copy(adj_col.at[pl.ds(jj, CH)], col_bufs[b], se.at[b])
            pltpu.async_copy(adj_row.at[pl.ds(jj, CH)], row_bufs[b], se.at[b])
            pltpu.async_copy(adj_val.at[pl.ds(jj, CH)], val_bufs[b], se.at[b])
        for b in range(QUAD):
            j = tchunk + k * QUAD + b
            jj = jnp.minimum(j, TOT_CH - 1) * CH
            pltpu.make_async_copy(adj_col.at[pl.ds(jj, CH)], col_bufs[b],
                                  se.at[b]).wait()
            pltpu.make_async_copy(adj_row.at[pl.ds(jj, CH)], row_bufs[b],
                                  se.at[b]).wait()
            pltpu.make_async_copy(adj_val.at[pl.ds(jj, CH)], val_bufs[b],
                                  se.at[b]).wait()

            @pl.when(j >= TOT_CH)
            def _():
                for g in range(CH // 16):
                    val_bufs[b][pl.ds(g * 16, 16)] = jnp.zeros(
                        (16,), jnp.float32)

            for g in range(CH // 16):
                col_bufs[b][pl.ds(g * 16, 16)] = (
                    col_bufs[b][pl.ds(g * 16, 16)] + coff)
            pltpu.async_copy(item_cat.at[col_bufs[b]], rows_bufs[b], sg.at[b])
        for b in range(QUAD):
            pltpu.make_async_copy(item_cat.at[col_bufs[b]], rows_bufs[b],
                                  sg.at[b]).wait()
            rb = rows_bufs[b]
            vb = val_bufs[b]
            for g in range(CH // 16):
                val16 = vb[pl.ds(g * 16, 16)]
                for t in range(16):
                    e = g * 16 + t
                    vv = jnp.broadcast_to(
                        lax.slice(val16, (t,), (t + 1,)), (16,))
                    rb[e, pl.ds(0, 16)] = rb[e, pl.ds(0, 16)] * vv
                    rb[e, pl.ds(16, 16)] = rb[e, pl.ds(16, 16)] * vv
            pltpu.async_copy(rb, acc_sh.at[row_bufs[b]], ss.at[b], add=True)
        return c

    lax.fori_loop(0, NCHUNK // QUAD, quad, 0)
    for b in range(QUAD):
        pltpu.make_async_copy(rows_bufs[b], acc_sh.at[bidx_v], ss.at[b]).wait()
    plsc.subcore_barrier()

    dbase = sid * PERT
    pltpu.sync_copy(
        acc_sh.at[pl.ds(dbase, PERT)],
        out_acc.at[pl.ds(dbase, PERT), pl.ds(cid * HALF, HALF)])


def _sc_batch_body(item_cat, users_emb, user_batch, item_batch, feats, dep,
                   out_ue, out_iblo, out_ibhi, out_fb,
                   rows0, bidx_v, bidx2_v, brow_v, bfeat_v, sg):
    cid = lax.axis_index("c")
    sid = lax.axis_index("s")
    wid = sid * NC + cid

    abase = wid * BPW
    pltpu.sync_copy(user_batch.at[pl.ds(abase, BPW)], bidx_v)
    for p in range(BPW // 32):
        pltpu.async_copy(users_emb.at[bidx_v.at[pl.ds(p * 32, 32)]],
                         brow_v, sg.at[0]).wait()
        pltpu.sync_copy(brow_v, out_ue.at[pl.ds(abase + p * 32, 32)])
    for p in range(BPW // 64):
        pltpu.async_copy(feats.at[bidx_v.at[pl.ds(p * 64, 64)]],
                         bfeat_v, sg.at[0]).wait()
        pltpu.sync_copy(bfeat_v, out_fb.at[pl.ds(abase + p * 64, 64)])
    pltpu.sync_copy(item_batch.at[pl.ds(abase, BPW)], bidx_v)
    off16 = jnp.full((16,), IN_, jnp.int32)
    for g in range(BPW // 16):
        bidx2_v[pl.ds(g * 16, 16)] = bidx_v[pl.ds(g * 16, 16)] + off16
    pltpu.async_copy(item_cat.at[bidx_v], rows0, sg.at[0]).wait()
    pltpu.sync_copy(rows0, out_iblo.at[pl.ds(abase, BPW)])
    pltpu.async_copy(item_cat.at[bidx2_v], rows0, sg.at[0]).wait()
    pltpu.sync_copy(rows0, out_ibhi.at[pl.ds(abase, BPW)])


def _sc_mesh():
    return plsc.VectorSubcoreMesh(core_axis_name="c", subcore_axis_name="s",
                                  num_cores=NC, num_subcores=NS)


def _sc_edges(adj_row, adj_col, adj_val, item_cat):
    f = functools.partial(
        pl.kernel,
        out_type=jax.ShapeDtypeStruct((UNP, 128), jnp.float32),
        mesh=_sc_mesh(),
        scratch_types=[
            pltpu.VMEM((CH, HALF), jnp.float32),
            pltpu.VMEM((CH, HALF), jnp.float32),
            pltpu.VMEM((CH, HALF), jnp.float32),
            pltpu.VMEM((CH, HALF), jnp.float32),
            pltpu.VMEM((CH,), jnp.int32),
            pltpu.VMEM((CH,), jnp.int32),
            pltpu.VMEM((CH,), jnp.int32),
            pltpu.VMEM((CH,), jnp.int32),
            pltpu.VMEM((CH,), jnp.int32),
            pltpu.VMEM((CH,), jnp.int32),
            pltpu.VMEM((CH,), jnp.int32),
            pltpu.VMEM((CH,), jnp.int32),
            pltpu.VMEM((CH,), jnp.float32),
            pltpu.VMEM((CH,), jnp.float32),
            pltpu.VMEM((CH,), jnp.float32),
            pltpu.VMEM((CH,), jnp.float32),
            pltpu.VMEM((CH,), jnp.int32),
            pltpu.VMEM_SHARED((UNP, HALF), jnp.float32),
            pltpu.SemaphoreType.DMA((QUAD,)),
            pltpu.SemaphoreType.DMA((QUAD,)),
            pltpu.SemaphoreType.DMA((QUAD,)),
        ],
        compiler_params=pltpu.CompilerParams(use_tc_tiling_on_sc=False),
    )(_sc_edges_body)
    return f(adj_row, adj_col, adj_val, item_cat)


def _sc_batch(item_cat, users_emb, user_batch, item_batch, feats_p, dep):
    f = functools.partial(
        pl.kernel,
        out_type=(
            jax.ShapeDtypeStruct((BN, F), jnp.float32),
            jax.ShapeDtypeStruct((BN, HALF), jnp.float32),
            jax.ShapeDtypeStruct((BN, HALF), jnp.float32),
            jax.ShapeDtypeStruct((BN, 16), jnp.int32),
        ),
        mesh=_sc_mesh(),
        scratch_types=[
            pltpu.VMEM((CH, HALF), jnp.float32),
            pltpu.VMEM((BPW,), jnp.int32),
            pltpu.VMEM((BPW,), jnp.int32),
            pltpu.VMEM((32, F), jnp.float32),
            pltpu.VMEM((64, 16), jnp.int32),
            pltpu.SemaphoreType.DMA((1,)),
        ],
        compiler_params=pltpu.CompilerParams(use_tc_tiling_on_sc=False),
    )(_sc_batch_body)
    return f(item_cat, users_emb, user_batch, item_batch, feats_p, dep)


def kernel(adj_row, adj_col, adj_val, user_batch, rating_batch, item_batch,
           flag_t, users_features, gcn_users_embedding0, gcn_items_embedding0,
           f1_w1, f1_b1, f1_w2, f1_b2, f2_w1, f2_b1, f2_w2, f2_b2,
           d1_w1, d1_b1, d1_w2, d1_b2, d2_w1, d2_b1, d2_w2, d2_b2):
    w1c = jnp.concatenate([f1_w1, f2_w1], axis=1)
    b1c = jnp.concatenate([f1_b1, f2_b1])[None, :]
    w2c = jnp.concatenate([f1_w2, f2_w2], axis=0)
    b2s = (f1_b2 + f2_b2)[None, :]
    wd1 = jnp.concatenate([d1_w1, d2_w1], axis=1)
    bd1 = jnp.concatenate([d1_b1, d2_b1])[None, :]
    wd2 = jnp.zeros((2 * F, 128), jnp.float32)
    wd2 = wd2.at[:F, 0:2].set(d1_w2).at[F:, 2:5].set(d2_w2)
    bd2 = jnp.zeros((128,), jnp.float32)
    bd2 = bd2.at[0:2].set(d1_b2).at[2:5].set(d2_b2)[None, :]

    feats_p = jnp.pad(users_features, ((0, UNP - UN), (0, 14)))

    item_lo, item_hi = _item_mlp(gcn_items_embedding0, w1c, b1c, w2c, b2s)
    item_cat = jnp.concatenate([item_lo, item_hi], axis=0)

    acc = _sc_edges(adj_row, adj_col, adj_val, item_cat)
    ue, ib_lo, ib_hi, fb = _sc_batch(
        item_cat, gcn_users_embedding0, user_batch, item_batch, feats_p,
        acc)

    ib = jnp.concatenate([ib_lo, ib_hi], axis=1)
    rb2 = rating_batch[:, None]

    bout = _batch_losses(ue, fb, ib, rb2, w1c, b1c, w2c, b2s,
                         wd1, bd1, wd2, bd2)
    lout = _local_losses(acc, feats_p, wd1, bd1, wd2, bd2)

    d_loss1 = bout[0, 0]
    d_loss2 = bout[0, 1]
    lps = bout[0, 2]
    d_loss1_local = lout[0, 0]
    d_loss2_local = lout[0, 1]

    d_loss = (d_loss1 * 2.0 + d_loss2) / 2.0
    d_loss_local = d_loss1_local * 2.0 + d_loss2_local
    d_loss_all = 10.0 * (d_loss + 0.5 * d_loss_local)
    g_loss_all = 0.1 * lps - d_loss_all
    g_d_loss_all = -d_loss_all
    return (d_loss_all, g_loss_all, g_d_loss_all)


# speedup vs baseline: 17.8622x; 1.0131x over previous
import functools

import jax
import jax.numpy as jnp
from jax import lax
from jax.experimental import pallas as pl
from jax.experimental.pallas import tpu as pltpu
from jax.experimental.pallas import tpu_sc as plsc

UN = 50000
IN_ = 20000
F = 64
BN = 4096
E = 800000

NC, NS = 2, 16
NW = NC * NS
BPW = BN // NW
CH = 128
TOT_CH = E // CH
NCHUNK = 392
UNP = 50048
PERT = UNP // NS
HALF = F // 2


def _leaky(x):
    return jnp.where(x > 0, x, 0.01 * x)


def _item_mlp_body(x_ref, w1_ref, b1_ref, w2_ref, b2_ref, o_ref):
    x = x_ref[...]
    h = _leaky(jnp.dot(x, w1_ref[...], preferred_element_type=jnp.float32)
               + b1_ref[...])
    o_ref[...] = (jnp.dot(h, w2_ref[...], preferred_element_type=jnp.float32)
                  + b2_ref[...]) * 0.5


def _item_mlp(items, w1c, b1c, w2c, b2s):
    blk = 2000
    grid = IN_ // blk
    return pl.pallas_call(
        _item_mlp_body,
        grid=(grid,),
        in_specs=[
            pl.BlockSpec((blk, F), lambda i: (i, 0)),
            pl.BlockSpec((F, 4 * F), lambda i: (0, 0)),
            pl.BlockSpec((1, 4 * F), lambda i: (0, 0)),
            pl.BlockSpec((4 * F, F), lambda i: (0, 0)),
            pl.BlockSpec((1, F), lambda i: (0, 0)),
        ],
        out_specs=pl.BlockSpec((blk, F), lambda i: (i, 0)),
        out_shape=jax.ShapeDtypeStruct((IN_, F), jnp.float32),
    )(items, w1c, b1c, w2c, b2s)


def _batch_body(ue_ref, fb_ref, ib_ref, rb_ref, w1_ref, b1_ref, w2_ref,
                b2_ref, wd1_ref, bd1_ref, wd2_ref, bd2_ref, out_ref):
    x = ue_ref[...]
    h = _leaky(jnp.dot(x, w1_ref[...], preferred_element_type=jnp.float32)
               + b1_ref[...])
    ub = (jnp.dot(h, w2_ref[...], preferred_element_type=jnp.float32)
          + b2_ref[...]) * 0.5
    hd = _leaky(jnp.dot(ub, wd1_ref[...], preferred_element_type=jnp.float32)
                + bd1_ref[...])
    logits = (jnp.dot(hd, wd2_ref[...], preferred_element_type=jnp.float32)
              + bd2_ref[...])
    col = lax.broadcasted_iota(jnp.int32, logits.shape, 1)
    neg = jnp.float32(-1e30)
    lse1 = jnp.log(jnp.sum(jnp.exp(jnp.where(col < 2, logits, neg)), -1))
    lse2 = jnp.log(jnp.sum(
        jnp.exp(jnp.where((col >= 2) & (col < 5), logits, neg)), -1))
    gender = fb_ref[...][:, 0:1]
    age = fb_ref[...][:, 1:2]
    ll1 = jnp.sum(jnp.where(col == gender, logits, 0.0), -1)
    ll2 = jnp.sum(jnp.where(col == 2 + age, logits, 0.0), -1)
    d_loss1 = jnp.mean(lse1 - ll1)
    d_loss2 = jnp.mean(lse2 - ll2)
    ib = ib_ref[...]
    pred = jnp.sum(ub * ib, -1)
    loss_part = jnp.mean((pred - rb_ref[...][:, 0]) ** 2)
    l2 = 0.001 * jnp.mean(jnp.sum(ub * ub + ib * ib, -1))
    lps = loss_part + l2
    ocol = lax.broadcasted_iota(jnp.int32, (1, 128), 1)
    out_ref[...] = (jnp.where(ocol == 0, d_loss1, 0.0)
                    + jnp.where(ocol == 1, d_loss2, 0.0)
                    + jnp.where(ocol == 2, lps, 0.0))


def _batch_losses(ue, fb, ib, rb2, w1c, b1c, w2c, b2s, wd1, bd1, wd2, bd2):
    return pl.pallas_call(
        _batch_body,
        in_specs=[pl.BlockSpec(a.shape, lambda: tuple(0 for _ in a.shape))
                  for a in (ue, fb, ib, rb2, w1c, b1c, w2c, b2s, wd1, bd1,
                            wd2, bd2)],
        out_specs=pl.BlockSpec((1, 128), lambda: (0, 0)),
        out_shape=jax.ShapeDtypeStruct((1, 128), jnp.float32),
    )(ue, fb, ib, rb2, w1c, b1c, w2c, b2s, wd1, bd1, wd2, bd2)


_LBLK = 2944


def _local_body(acc_ref, ft_ref, wd1_ref, bd1_ref, wd2_ref, bd2_ref,
                out_ref):
    i = pl.program_id(0)
    ng = pl.num_programs(0)
    blk = _LBLK
    x = lax.slice(acc_ref[...], (0, 0), (blk, F))
    hd = _leaky(jnp.dot(x, wd1_ref[...], preferred_element_type=jnp.float32)
                + bd1_ref[...])
    logits = (jnp.dot(hd, wd2_ref[...], preferred_element_type=jnp.float32)
              + bd2_ref[...])
    lt = logits.T
    l0 = lax.slice(lt, (0, 0), (1, blk))
    l1 = lax.slice(lt, (1, 0), (2, blk))
    l2 = lax.slice(lt, (2, 0), (3, blk))
    l3 = lax.slice(lt, (3, 0), (4, blk))
    l4 = lax.slice(lt, (4, 0), (5, blk))
    lse1 = jnp.log(jnp.exp(l0) + jnp.exp(l1))
    lse2 = jnp.log(jnp.exp(l2) + jnp.exp(l3) + jnp.exp(l4))
    ftt = ft_ref[...].astype(jnp.float32).T
    g = lax.slice(ftt, (0, 0), (1, blk))
    a = lax.slice(ftt, (1, 0), (2, blk))
    ll1 = l0 * (1.0 - g) + l1 * g
    ll2 = (l2 * (a == 0).astype(jnp.float32)
           + l3 * (a == 1).astype(jnp.float32)
           + l4 * (a == 2).astype(jnp.float32))
    ridx = i * blk + lax.broadcasted_iota(jnp.int32, (1, blk), 1)
    valid = (ridx < UN).astype(jnp.float32)
    s1 = jnp.sum((lse1 - ll1) * valid)
    s2 = jnp.sum((lse2 - ll2) * valid)
    ocol = lax.broadcasted_iota(jnp.int32, (1, 128), 1)
    part = (jnp.where(ocol == 0, s1, 0.0) + jnp.where(ocol == 1, s2, 0.0))

    @pl.when(i == 0)
    def _():
        out_ref[...] = jnp.zeros_like(out_ref)

    out_ref[...] += part

    @pl.when(i == ng - 1)
    def _():
        out_ref[...] = out_ref[...] * (1.0 / UN)


def _local_losses(acc, feats, wd1, bd1, wd2, bd2):
    blk = _LBLK
    grid = UNP // blk
    return pl.pallas_call(
        _local_body,
        grid=(grid,),
        in_specs=[
            pl.BlockSpec((blk, 128), lambda i: (i, 0)),
            pl.BlockSpec((blk, 16), lambda i: (i, 0)),
            pl.BlockSpec((F, 2 * F), lambda i: (0, 0)),
            pl.BlockSpec((1, 2 * F), lambda i: (0, 0)),
            pl.BlockSpec((2 * F, 128), lambda i: (0, 0)),
            pl.BlockSpec((1, 128), lambda i: (0, 0)),
        ],
        out_specs=pl.BlockSpec((1, 128), lambda i: (0, 0)),
        out_shape=jax.ShapeDtypeStruct((1, 128), jnp.float32),
    )(acc, feats, wd1, bd1, wd2, bd2)


QUAD = 4


def _sc_edges_body(adj_row, adj_col, adj_val, item_cat, out_acc, *scr):
    cid = lax.axis_index("c")
    sid = lax.axis_index("s")
    rows_bufs = scr[0:QUAD]
    col_bufs = scr[QUAD:2 * QUAD]
    row_bufs = scr[2 * QUAD:3 * QUAD]
    val_bufs = scr[3 * QUAD:4 * QUAD]
    bidx_v = scr[4 * QUAD]
    acc_sh = scr[4 * QUAD + 1]
    sg = scr[4 * QUAD + 2]
    ss = scr[4 * QUAD + 3]
    se = scr[4 * QUAD + 4]
    rows0 = rows_bufs[0]

    zeros16 = jnp.zeros((16,), jnp.float32)
    for i in range(CH):
        rows0[i, pl.ds(0, 16)] = zeros16
        rows0[i, pl.ds(16, 16)] = zeros16
    for g in range(BPW // 16):
        bidx_v[pl.ds(g * 16, 16)] = jnp.zeros((16,), jnp.int32)
    zbase = sid * PERT
    nfull = PERT // CH
    rem = PERT - nfull * CH
    for z in range(nfull):
        pltpu.sync_copy(rows0, acc_sh.at[pl.ds(zbase + z * CH, CH)])
    pltpu.sync_copy(rows0.at[pl.ds(0, rem)],
                    acc_sh.at[pl.ds(zbase + nfull * CH, rem)])
    plsc.subcore_barrier()

    coff = jnp.broadcast_to(cid * IN_, (16,)).astype(jnp.int32)
    tchunk = sid * NCHUNK

    for b in range(1, QUAD):
        pltpu.sync_copy(acc_sh.at[pl.ds(zbase, CH)], rows_bufs[b])
    for b in range(QUAD):
        pltpu.async_copy(rows_bufs[b], acc_sh.at[bidx_v], ss.at[b], add=True)

    def quad(k, c):
        for b in range(QUAD):
            j = tchunk + k * QUAD + b
            jj = jnp.minimum(j, TOT_CH - 1) * CH
            pltpu.make_async_copy(rows_bufs[b], acc_sh.at[bidx_v],
                                  ss.at[b]).wait()
            pltpu.async_copy(adj_col.at[pl.ds(jj, CH)], col_bufs[b], se.at[b])
            pltpu.async_copy(adj_row.at[pl.ds(jj, CH)], row_bufs[b], se.at[b])
            pltpu.async_copy(adj_val.at[pl.ds(jj, CH)], val_bufs[b], se.at[b])
        for b in range(QUAD):
            j = tchunk + k * QUAD + b
            jj = jnp.minimum(j, TOT_CH - 1) * CH
            pltpu.make_async_copy(adj_col.at[pl.ds(jj, CH)], col_bufs[b],
                                  se.at[b]).wait()
            pltpu.make_async_copy(adj_row.at[pl.ds(jj, CH)], row_bufs[b],
                                  se.at[b]).wait()
            pltpu.make_async_copy(adj_val.at[pl.ds(jj, CH)], val_bufs[b],
                                  se.at[b]).wait()

            @pl.when(j >= TOT_CH)
            def _():
                for g in range(CH // 16):
                    val_bufs[b][pl.ds(g * 16, 16)] = jnp.zeros(
                        (16,), jnp.float32)

            for g in range(CH // 16):
                col_bufs[b][pl.ds(g * 16, 16)] = (
                    col_bufs[b][pl.ds(g * 16, 16)] + coff)
            pltpu.async_copy(item_cat.at[col_bufs[b]], rows_bufs[b], sg.at[b])
        for b in range(QUAD):
            pltpu.make_async_copy(item_cat.at[col_bufs[b]], rows_bufs[b],
                                  sg.at[b]).wait()
            rb = rows_bufs[b]
            vb = val_bufs[b]
            for g in range(CH // 16):
                val16 = vb[pl.ds(g * 16, 16)]
                for t in range(16):
                    e = g * 16 + t
                    vv = jnp.broadcast_to(
                        lax.slice(val16, (t,), (t + 1,)), (16,))
                    rb[e, pl.ds(0, 16)] = rb[e, pl.ds(0, 16)] * vv
                    rb[e, pl.ds(16, 16)] = rb[e, pl.ds(16, 16)] * vv
            pltpu.async_copy(rb, acc_sh.at[row_bufs[b]], ss.at[b], add=True)
        return c

    lax.fori_loop(0, NCHUNK // QUAD, quad, 0)
    for b in range(QUAD):
        pltpu.make_async_copy(rows_bufs[b], acc_sh.at[bidx_v], ss.at[b]).wait()
    plsc.subcore_barrier()

    dbase = sid * PERT
    pltpu.sync_copy(
        acc_sh.at[pl.ds(dbase, PERT)],
        out_acc.at[pl.ds(dbase, PERT), pl.ds(cid * HALF, HALF)])


def _sc_batch_body(item_cat, users_emb, user_batch, item_batch, feats, dep,
                   out_ue, out_iblo, out_ibhi, out_fb,
                   rows0, bidx_v, bidx2_v, brow_v, bfeat_v, sg):
    cid = lax.axis_index("c")
    sid = lax.axis_index("s")
    wid = sid * NC + cid

    abase = wid * BPW
    pltpu.sync_copy(user_batch.at[pl.ds(abase, BPW)], bidx_v)
    for p in range(BPW // 32):
        pltpu.async_copy(users_emb.at[bidx_v.at[pl.ds(p * 32, 32)]],
                         brow_v, sg.at[0]).wait()
        pltpu.sync_copy(brow_v, out_ue.at[pl.ds(abase + p * 32, 32)])
    for p in range(BPW // 64):
        pltpu.async_copy(feats.at[bidx_v.at[pl.ds(p * 64, 64)]],
                         bfeat_v, sg.at[0]).wait()
        pltpu.sync_copy(bfeat_v, out_fb.at[pl.ds(abase + p * 64, 64)])
    pltpu.sync_copy(item_batch.at[pl.ds(abase, BPW)], bidx_v)
    off16 = jnp.full((16,), IN_, jnp.int32)
    for g in range(BPW // 16):
        bidx2_v[pl.ds(g * 16, 16)] = bidx_v[pl.ds(g * 16, 16)] + off16
    pltpu.async_copy(item_cat.at[bidx_v], rows0, sg.at[0]).wait()
    pltpu.sync_copy(rows0, out_iblo.at[pl.ds(abase, BPW)])
    pltpu.async_copy(item_cat.at[bidx2_v], rows0, sg.at[0]).wait()
    pltpu.sync_copy(rows0, out_ibhi.at[pl.ds(abase, BPW)])


def _sc_mesh():
    return plsc.VectorSubcoreMesh(core_axis_name="c", subcore_axis_name="s",
                                  num_cores=NC, num_subcores=NS)


def _sc_edges(adj_row, adj_col, adj_val, item_cat):
    f = functools.partial(
        pl.kernel,
        out_type=jax.ShapeDtypeStruct((UNP, 128), jnp.float32),
        mesh=_sc_mesh(),
        scratch_types=(
            [pltpu.VMEM((CH, HALF), jnp.float32)] * QUAD
            + [pltpu.VMEM((CH,), jnp.int32)] * QUAD
            + [pltpu.VMEM((CH,), jnp.int32)] * QUAD
            + [pltpu.VMEM((CH,), jnp.float32)] * QUAD
            + [
                pltpu.VMEM((CH,), jnp.int32),
                pltpu.VMEM_SHARED((UNP, HALF), jnp.float32),
                pltpu.SemaphoreType.DMA((QUAD,)),
                pltpu.SemaphoreType.DMA((QUAD,)),
                pltpu.SemaphoreType.DMA((QUAD,)),
            ]
        ),
        compiler_params=pltpu.CompilerParams(use_tc_tiling_on_sc=False),
    )(_sc_edges_body)
    return f(adj_row, adj_col, adj_val, item_cat)


def _sc_batch(item_cat, users_emb, user_batch, item_batch, feats_p, dep):
    f = functools.partial(
        pl.kernel,
        out_type=(
            jax.ShapeDtypeStruct((BN, F), jnp.float32),
            jax.ShapeDtypeStruct((BN, HALF), jnp.float32),
            jax.ShapeDtypeStruct((BN, HALF), jnp.float32),
            jax.ShapeDtypeStruct((BN, 16), jnp.int32),
        ),
        mesh=_sc_mesh(),
        scratch_types=[
            pltpu.VMEM((CH, HALF), jnp.float32),
            pltpu.VMEM((BPW,), jnp.int32),
            pltpu.VMEM((BPW,), jnp.int32),
            pltpu.VMEM((32, F), jnp.float32),
            pltpu.VMEM((64, 16), jnp.int32),
            pltpu.SemaphoreType.DMA((1,)),
        ],
        compiler_params=pltpu.CompilerParams(use_tc_tiling_on_sc=False),
    )(_sc_batch_body)
    return f(item_cat, users_emb, user_batch, item_batch, feats_p, dep)


def kernel(adj_row, adj_col, adj_val, user_batch, rating_batch, item_batch,
           flag_t, users_features, gcn_users_embedding0, gcn_items_embedding0,
           f1_w1, f1_b1, f1_w2, f1_b2, f2_w1, f2_b1, f2_w2, f2_b2,
           d1_w1, d1_b1, d1_w2, d1_b2, d2_w1, d2_b1, d2_w2, d2_b2):
    w1c = jnp.concatenate([f1_w1, f2_w1], axis=1)
    b1c = jnp.concatenate([f1_b1, f2_b1])[None, :]
    w2c = jnp.concatenate([f1_w2, f2_w2], axis=0)
    b2s = (f1_b2 + f2_b2)[None, :]
    wd1 = jnp.concatenate([d1_w1, d2_w1], axis=1)
    bd1 = jnp.concatenate([d1_b1, d2_b1])[None, :]
    wd2 = jnp.zeros((2 * F, 128), jnp.float32)
    wd2 = wd2.at[:F, 0:2].set(d1_w2).at[F:, 2:5].set(d2_w2)
    bd2 = jnp.zeros((128,), jnp.float32)
    bd2 = bd2.at[0:2].set(d1_b2).at[2:5].set(d2_b2)[None, :]

    feats_p = jnp.pad(users_features, ((0, UNP - UN), (0, 14)))

    item_f = _item_mlp(gcn_items_embedding0, w1c, b1c, w2c, b2s)
    item_cat = jnp.concatenate(
        [item_f[:, :F // 2], item_f[:, F // 2:]], axis=0)

    acc = _sc_edges(adj_row, adj_col, adj_val, item_cat)
    ue, ib_lo, ib_hi, fb = _sc_batch(
        item_cat, gcn_users_embedding0, user_batch, item_batch, feats_p,
        acc)

    ib = jnp.concatenate([ib_lo, ib_hi], axis=1)
    rb2 = rating_batch[:, None]

    bout = _batch_losses(ue, fb, ib, rb2, w1c, b1c, w2c, b2s,
                         wd1, bd1, wd2, bd2)
    lout = _local_losses(acc, feats_p, wd1, bd1, wd2, bd2)

    d_loss1 = bout[0, 0]
    d_loss2 = bout[0, 1]
    lps = bout[0, 2]
    d_loss1_local = lout[0, 0]
    d_loss2_local = lout[0, 1]

    d_loss = (d_loss1 * 2.0 + d_loss2) / 2.0
    d_loss_local = d_loss1_local * 2.0 + d_loss2_local
    d_loss_all = 10.0 * (d_loss + 0.5 * d_loss_local)
    g_loss_all = 0.1 * lps - d_loss_all
    g_d_loss_all = -d_loss_all
    return (d_loss_all, g_loss_all, g_d_loss_all)
